# Initial kernel scaffold; baseline (speedup 1.0000x reference)
import functools

import jax
import jax.numpy as jnp
from jax import lax
from jax.experimental import pallas as pl
from jax.experimental.pallas import tpu as pltpu
from jax.experimental.pallas import tpu_sc as plsc

N = 10000
D = 128
E = 320000
NPAD = 10240
NT = 32
B = 128
NBLK = -(-E // (NT * B))
SLOTS = NT * NBLK * B
DUMMY = N + 16
RPT = NPAD // 16
DEG_D = 16
RBLK = 256


def _make_agg(d):
    mesh = plsc.VectorSubcoreMesh(core_axis_name="c", subcore_axis_name="s")

    @functools.partial(
        pl.kernel,
        out_type=jax.ShapeDtypeStruct((2, NPAD, d), jnp.float32),
        mesh=mesh,
        scratch_types=[
            pltpu.VMEM((NBLK, B), jnp.int32),
            pltpu.VMEM((NBLK, B), jnp.int32),
            pltpu.VMEM((B, d), jnp.float32),
            pltpu.VMEM((128, d), jnp.float32),
            pltpu.MemorySpace.VMEM_SHARED((NPAD, d), jnp.float32),
            pltpu.SemaphoreType.DMA,
        ],
    )
    def agg(g_hbm, cidx_hbm, ridx_hbm, zero_hbm, out_hbm,
            cidx_v, ridx_v, rows_v, zbuf_v, acc_sh, sem):
        c = lax.axis_index("c")
        s = lax.axis_index("s")
        wid = c * 16 + s
        base = s * RPT
        pltpu.sync_copy(zero_hbm, zbuf_v)
        for k in range(RPT // 128):
            pltpu.sync_copy(zbuf_v, acc_sh.at[pl.ds(base + k * 128, 128)])
        pltpu.sync_copy(cidx_hbm.at[wid], cidx_v)
        pltpu.sync_copy(ridx_hbm.at[wid], ridx_v)
        plsc.subcore_barrier()

        @pl.loop(0, NBLK)
        def _(j):
            pltpu.async_copy(g_hbm.at[cidx_v.at[j]], rows_v, sem).wait()
            pltpu.sync_copy(rows_v, acc_sh.at[ridx_v.at[j]], add=True)

        plsc.subcore_barrier()
        pltpu.sync_copy(acc_sh.at[pl.ds(base, RPT)],
                        out_hbm.at[c].at[pl.ds(base, RPT)])

    return agg


_agg_feat = _make_agg(D)
_agg_deg = _make_agg(DEG_D)


def _prep_body(x_ref, dg_ref, dis_ref, g0_ref):
    deg = dg_ref[0][:, :1] + dg_ref[1][:, :1] + 1.0
    disb = jnp.broadcast_to(lax.rsqrt(deg), x_ref.shape)
    dis_ref[...] = disb
    g0_ref[...] = disb * x_ref[...]


def _prep(xp, deg2):
    grid = (NPAD // RBLK,)
    return pl.pallas_call(
        _prep_body,
        grid=grid,
        in_specs=[
            pl.BlockSpec((RBLK, D), lambda i: (i, 0)),
            pl.BlockSpec((2, RBLK, DEG_D), lambda i: (0, i, 0)),
        ],
        out_specs=[
            pl.BlockSpec((RBLK, D), lambda i: (i, 0)),
            pl.BlockSpec((RBLK, D), lambda i: (i, 0)),
        ],
        out_shape=[
            jax.ShapeDtypeStruct((NPAD, D), jnp.float32),
            jax.ShapeDtypeStruct((NPAD, D), jnp.float32),
        ],
    )(xp, deg2)


def _layer_body(final, a_ref, g_ref, dis_ref, w_ref, b_ref, o_ref):
    t = dis_ref[...] * (a_ref[0] + a_ref[1] + g_ref[...])
    h = jnp.dot(t, w_ref[...], preferred_element_type=jnp.float32) + b_ref[...]
    if final:
        o_ref[...] = h
    else:
        o_ref[...] = dis_ref[...] * jnp.maximum(h, 0.0)


def _layer(a2, g, dis, w, b, final):
    grid = (NPAD // RBLK,)
    return pl.pallas_call(
        functools.partial(_layer_body, final),
        grid=grid,
        in_specs=[
            pl.BlockSpec((2, RBLK, D), lambda i: (0, i, 0)),
            pl.BlockSpec((RBLK, D), lambda i: (i, 0)),
            pl.BlockSpec((RBLK, D), lambda i: (i, 0)),
            pl.BlockSpec((D, D), lambda i: (0, 0)),
            pl.BlockSpec((1, D), lambda i: (0, 0)),
        ],
        out_specs=pl.BlockSpec((RBLK, D), lambda i: (i, 0)),
        out_shape=jax.ShapeDtypeStruct((NPAD, D), jnp.float32),
    )(a2, g, dis, w, b)


def kernel(x, edge_index, W1, b1, W2, b2, W3, b3):
    xp = jnp.pad(x, ((0, NPAD - N), (0, 0)))
    row = edge_index[0].astype(jnp.int32)
    col = edge_index[1].astype(jnp.int32)
    pad = SLOTS - E
    ridx = jnp.concatenate(
        [row, jnp.full((pad,), DUMMY, jnp.int32)]).reshape(NT, NBLK, B)
    cidx = jnp.concatenate(
        [col, jnp.zeros((pad,), jnp.int32)]).reshape(NT, NBLK, B)
    zeros_d = jnp.zeros((128, D), jnp.float32)
    zeros_g = jnp.zeros((128, DEG_D), jnp.float32)
    ones_g = jnp.ones((NPAD, DEG_D), jnp.float32)

    deg2 = _agg_deg(ones_g, cidx, ridx, zeros_g)
    dis, g0 = _prep(xp, deg2)

    a1 = _agg_feat(g0, cidx, ridx, zeros_d)
    g1 = _layer(a1, g0, dis, W1, b1.reshape(1, D), final=False)
    a2 = _agg_feat(g1, cidx, ridx, zeros_d)
    g2 = _layer(a2, g1, dis, W2, b2.reshape(1, D), final=False)
    a3 = _agg_feat(g2, cidx, ridx, zeros_d)
    out = _layer(a3, g2, dis, W3, b3.reshape(1, D), final=True)
    return out[:N]


# speedup vs baseline: 5.4850x; 5.4850x over previous
import functools

import jax
import jax.numpy as jnp
from jax import lax
from jax.experimental import pallas as pl
from jax.experimental.pallas import tpu as pltpu
from jax.experimental.pallas import tpu_sc as plsc

N = 10000
D = 128
E = 320000
NPAD = 10240
NT = 32
B = 128
NBLK = 80
SLOTS = NT * NBLK * B
DUMMY = N + 16
RPT = NPAD // 16
RBLK = 256
NRB = NPAD // RBLK


def _make_agg(d):
    mesh = plsc.VectorSubcoreMesh(core_axis_name="c", subcore_axis_name="s")

    @functools.partial(
        pl.kernel,
        out_type=jax.ShapeDtypeStruct((2 * NPAD, d), jnp.float32),
        mesh=mesh,
        scratch_types=[
            pltpu.VMEM((NBLK, B), jnp.int32),
            pltpu.VMEM((NBLK, B), jnp.int32),
            pltpu.VMEM((B, d), jnp.float32),
            pltpu.VMEM((64, d), jnp.float32),
            pltpu.MemorySpace.VMEM_SHARED((NPAD, d), jnp.float32),
            pltpu.SemaphoreType.DMA,
        ],
    )
    def agg(g_hbm, cidx_hbm, ridx_hbm, zero_hbm, out_hbm,
            cidx_v, ridx_v, rows_v, zbuf_v, acc_sh, sem):
        c = lax.axis_index("c")
        s = lax.axis_index("s")
        wid = c * 16 + s
        base = s * RPT
        pltpu.sync_copy(zero_hbm, zbuf_v)
        for k in range(RPT // 64):
            pltpu.sync_copy(zbuf_v, acc_sh.at[pl.ds(base + k * 64, 64)])
        pltpu.sync_copy(cidx_hbm.at[pl.ds(wid * NBLK, NBLK)], cidx_v)
        pltpu.sync_copy(ridx_hbm.at[pl.ds(wid * NBLK, NBLK)], ridx_v)
        plsc.subcore_barrier()

        @pl.loop(0, NBLK)
        def _(j):
            pltpu.async_copy(g_hbm.at[cidx_v.at[j]], rows_v, sem).wait()
            pltpu.sync_copy(rows_v, acc_sh.at[ridx_v.at[j]], add=True)

        plsc.subcore_barrier()
        pltpu.sync_copy(acc_sh.at[pl.ds(base, RPT)],
                        out_hbm.at[pl.ds(c * NPAD + base, RPT)])

    return agg


_agg_feat = _make_agg(D)


def _prep_body(x_ref, da_ref, db_ref, dis_ref, g0_ref):
    deg = da_ref[:, :1] + db_ref[:, :1] + 1.0
    disb = jnp.broadcast_to(lax.rsqrt(deg), x_ref.shape)
    dis_ref[...] = disb
    g0_ref[...] = disb * x_ref[...]


def _prep(xp, deg2):
    return pl.pallas_call(
        _prep_body,
        grid=(NRB,),
        in_specs=[
            pl.BlockSpec((RBLK, D), lambda i: (i, 0)),
            pl.BlockSpec((RBLK, D), lambda i: (i, 0)),
            pl.BlockSpec((RBLK, D), lambda i: (i + NRB, 0)),
        ],
        out_specs=[
            pl.BlockSpec((RBLK, D), lambda i: (i, 0)),
            pl.BlockSpec((RBLK, D), lambda i: (i, 0)),
        ],
        out_shape=[
            jax.ShapeDtypeStruct((NPAD, D), jnp.float32),
            jax.ShapeDtypeStruct((NPAD, D), jnp.float32),
        ],
    )(xp, deg2, deg2)


def _layer_body(final, aa_ref, ab_ref, g_ref, dis_ref, w_ref, b_ref, o_ref):
    t = dis_ref[...] * (aa_ref[...] + ab_ref[...] + g_ref[...])
    h = jnp.dot(t, w_ref[...], preferred_element_type=jnp.float32) + b_ref[...]
    if final:
        o_ref[...] = h
    else:
        o_ref[...] = dis_ref[...] * jnp.maximum(h, 0.0)


def _layer(a2, g, dis, w, b, final):
    return pl.pallas_call(
        functools.partial(_layer_body, final),
        grid=(NRB,),
        in_specs=[
            pl.BlockSpec((RBLK, D), lambda i: (i, 0)),
            pl.BlockSpec((RBLK, D), lambda i: (i + NRB, 0)),
            pl.BlockSpec((RBLK, D), lambda i: (i, 0)),
            pl.BlockSpec((RBLK, D), lambda i: (i, 0)),
            pl.BlockSpec((D, D), lambda i: (0, 0)),
            pl.BlockSpec((1, D), lambda i: (0, 0)),
        ],
        out_specs=pl.BlockSpec((RBLK, D), lambda i: (i, 0)),
        out_shape=jax.ShapeDtypeStruct((NPAD, D), jnp.float32),
    )(a2, a2, g, dis, w, b)


def kernel(x, edge_index, W1, b1, W2, b2, W3, b3):
    xp = jnp.pad(x, ((0, NPAD - N), (0, 0)))
    row = edge_index[0].astype(jnp.int32)
    col = edge_index[1].astype(jnp.int32)
    pad = SLOTS - E
    ridx = jnp.concatenate(
        [row, jnp.full((pad,), DUMMY, jnp.int32)]).reshape(NT * NBLK, B)
    cidx = jnp.concatenate(
        [col, jnp.zeros((pad,), jnp.int32)]).reshape(NT * NBLK, B)
    zeros_d = jnp.zeros((64, D), jnp.float32)
    ones_d = jnp.ones((NPAD, D), jnp.float32)

    deg2 = _agg_feat(ones_d, cidx, ridx, zeros_d)
    dis, g0 = _prep(xp, deg2)

    a1 = _agg_feat(g0, cidx, ridx, zeros_d)
    g1 = _layer(a1, g0, dis, W1, b1.reshape(1, D), final=False)
    a2 = _agg_feat(g1, cidx, ridx, zeros_d)
    g2 = _layer(a2, g1, dis, W2, b2.reshape(1, D), final=False)
    a3 = _agg_feat(g2, cidx, ridx, zeros_d)
    out = _layer(a3, g2, dis, W3, b3.reshape(1, D), final=True)
    return out[:N]


# speedup vs baseline: 16.0683x; 2.9295x over previous
import functools

import jax
import jax.numpy as jnp
from jax import lax
from jax.experimental import pallas as pl
from jax.experimental.pallas import tpu as pltpu
from jax.experimental.pallas import tpu_sc as plsc

N = 10000
D = 128
E = 320000
NPAD = 10240
NT = 32
B = 128
NBLK = 80
SLOTS = NT * NBLK * B
RPT = NPAD // 16
CB = 16
NCHUNK = NBLK // CB
RBLK = 256
NRB = NPAD // RBLK


def _make_agg(d):
    mesh = plsc.VectorSubcoreMesh(core_axis_name="c", subcore_axis_name="s")

    @functools.partial(
        pl.kernel,
        out_type=jax.ShapeDtypeStruct((2 * NPAD, d), jnp.float32),
        mesh=mesh,
        scratch_types=[
            pltpu.VMEM((CB, B), jnp.int32),
            pltpu.VMEM((CB, B), jnp.int32),
            pltpu.VMEM((B, d), jnp.float32),
            pltpu.VMEM((B, d), jnp.float32),
            pltpu.MemorySpace.VMEM_SHARED((NPAD, d), jnp.float32),
            pltpu.SemaphoreType.DMA,
            pltpu.SemaphoreType.DMA,
        ],
    )
    def agg(g_hbm, cidx_hbm, ridx_hbm, zero_hbm, out_hbm,
            cbuf, rbuf, rows0, rows1, acc_sh, gs0, gs1):
        c = lax.axis_index("c")
        s = lax.axis_index("s")
        wid = c * 16 + s
        base = s * RPT
        pltpu.sync_copy(zero_hbm, acc_sh.at[pl.ds(base, RPT)])
        plsc.subcore_barrier()

        @pl.loop(0, NCHUNK)
        def _(q):
            off = (wid * NCHUNK + q) * CB
            pltpu.sync_copy(cidx_hbm.at[pl.ds(off, CB)], cbuf)
            pltpu.sync_copy(ridx_hbm.at[pl.ds(off, CB)], rbuf)

            @pl.loop(0, CB, step=2)
            def _(j):
                a0 = pltpu.async_copy(g_hbm.at[cbuf.at[j]], rows0, gs0)
                a1 = pltpu.async_copy(g_hbm.at[cbuf.at[j + 1]], rows1, gs1)
                a0.wait()
                pltpu.sync_copy(rows0, acc_sh.at[rbuf.at[j]], add=True)
                a1.wait()
                pltpu.sync_copy(rows1, acc_sh.at[rbuf.at[j + 1]], add=True)

        plsc.subcore_barrier()
        pltpu.sync_copy(acc_sh.at[pl.ds(base, RPT)],
                        out_hbm.at[pl.ds(c * NPAD + base, RPT)])

    return agg


_agg_feat = _make_agg(D)


def _prep_body(x_ref, da_ref, db_ref, dis_ref, g0_ref):
    deg = da_ref[:, :1] + db_ref[:, :1] + 1.0
    disb = jnp.broadcast_to(lax.rsqrt(deg), x_ref.shape)
    dis_ref[...] = disb
    g0_ref[...] = disb * x_ref[...]


def _prep(xp, deg2):
    return pl.pallas_call(
        _prep_body,
        grid=(NRB,),
        in_specs=[
            pl.BlockSpec((RBLK, D), lambda i: (i, 0)),
            pl.BlockSpec((RBLK, D), lambda i: (i, 0)),
            pl.BlockSpec((RBLK, D), lambda i: (i + NRB, 0)),
        ],
        out_specs=[
            pl.BlockSpec((RBLK, D), lambda i: (i, 0)),
            pl.BlockSpec((RBLK, D), lambda i: (i, 0)),
        ],
        out_shape=[
            jax.ShapeDtypeStruct((NPAD, D), jnp.float32),
            jax.ShapeDtypeStruct((NPAD, D), jnp.float32),
        ],
    )(xp, deg2, deg2)


def _layer_body(final, aa_ref, ab_ref, g_ref, dis_ref, w_ref, b_ref, o_ref):
    t = dis_ref[...] * (aa_ref[...] + ab_ref[...] + g_ref[...])
    h = jnp.dot(t, w_ref[...], preferred_element_type=jnp.float32) + b_ref[...]
    if final:
        o_ref[...] = h
    else:
        o_ref[...] = dis_ref[...] * jnp.maximum(h, 0.0)


def _layer(a2, g, dis, w, b, final):
    return pl.pallas_call(
        functools.partial(_layer_body, final),
        grid=(NRB,),
        in_specs=[
            pl.BlockSpec((RBLK, D), lambda i: (i, 0)),
            pl.BlockSpec((RBLK, D), lambda i: (i + NRB, 0)),
            pl.BlockSpec((RBLK, D), lambda i: (i, 0)),
            pl.BlockSpec((RBLK, D), lambda i: (i, 0)),
            pl.BlockSpec((D, D), lambda i: (0, 0)),
            pl.BlockSpec((1, D), lambda i: (0, 0)),
        ],
        out_specs=pl.BlockSpec((RBLK, D), lambda i: (i, 0)),
        out_shape=jax.ShapeDtypeStruct((NPAD, D), jnp.float32),
    )(a2, a2, g, dis, w, b)


def kernel(x, edge_index, W1, b1, W2, b2, W3, b3):
    xp = jnp.pad(x, ((0, NPAD - N), (0, 0)))
    row = edge_index[0].astype(jnp.int32)
    col = edge_index[1].astype(jnp.int32)
    pad = SLOTS - E
    pr = N + jnp.arange(pad, dtype=jnp.int32) % (NPAD - N)
    pc = jnp.arange(pad, dtype=jnp.int32) % N
    ridx = jnp.concatenate([row, pr]).reshape(NT * NBLK, B)
    cidx = jnp.concatenate([col, pc]).reshape(NT * NBLK, B)
    zeros_d = jnp.zeros((RPT, D), jnp.float32)
    ones_d = jnp.ones((NPAD, D), jnp.float32)

    deg2 = _agg_feat(ones_d, cidx, ridx, zeros_d)
    dis, g0 = _prep(xp, deg2)

    a1 = _agg_feat(g0, cidx, ridx, zeros_d)
    g1 = _layer(a1, g0, dis, W1, b1.reshape(1, D), final=False)
    a2 = _agg_feat(g1, cidx, ridx, zeros_d)
    g2 = _layer(a2, g1, dis, W2, b2.reshape(1, D), final=False)
    a3 = _agg_feat(g2, cidx, ridx, zeros_d)
    out = _layer(a3, g2, dis, W3, b3.reshape(1, D), final=True)
    return out[:N]


# speedup vs baseline: 16.2236x; 1.0097x over previous
import functools

import jax
import jax.numpy as jnp
from jax import lax
from jax.experimental import pallas as pl
from jax.experimental.pallas import tpu as pltpu
from jax.experimental.pallas import tpu_sc as plsc

N = 10000
D = 128
E = 320000
NPAD = 10240
NT = 32
B = 128
NBLK = 80
SLOTS = NT * NBLK * B
RPT = NPAD // 16
CB = 16
NCHUNK = NBLK // CB
RBLK = 256
NRB = NPAD // RBLK


def _make_agg(d):
    mesh = plsc.VectorSubcoreMesh(core_axis_name="c", subcore_axis_name="s")

    @functools.partial(
        pl.kernel,
        out_type=jax.ShapeDtypeStruct((2 * NPAD, d), jnp.float32),
        mesh=mesh,
        scratch_types=[
            pltpu.VMEM((CB, B), jnp.int32),
            pltpu.VMEM((CB, B), jnp.int32),
            pltpu.VMEM((B, d), jnp.float32),
            pltpu.VMEM((B, d), jnp.float32),
            pltpu.MemorySpace.VMEM_SHARED((NPAD, d), jnp.float32),
            pltpu.SemaphoreType.DMA,
            pltpu.SemaphoreType.DMA,
            pltpu.SemaphoreType.DMA,
            pltpu.SemaphoreType.DMA,
        ],
    )
    def agg(g_hbm, cidx_hbm, ridx_hbm, zero_hbm, out_hbm,
            cbuf, rbuf, rows0, rows1, acc_sh, gs0, gs1, ss0, ss1):
        c = lax.axis_index("c")
        s = lax.axis_index("s")
        wid = c * 16 + s
        base = s * RPT
        pltpu.sync_copy(zero_hbm, acc_sh.at[pl.ds(base, RPT)])
        plsc.subcore_barrier()

        @pl.loop(0, NCHUNK)
        def _(q):
            off = (wid * NCHUNK + q) * CB
            pltpu.sync_copy(cidx_hbm.at[pl.ds(off, CB)], cbuf)
            pltpu.sync_copy(ridx_hbm.at[pl.ds(off, CB)], rbuf)

            @pl.loop(0, CB, step=2)
            def _(j):
                a0 = pltpu.async_copy(g_hbm.at[cbuf.at[j]], rows0, gs0)
                a1 = pltpu.async_copy(g_hbm.at[cbuf.at[j + 1]], rows1, gs1)
                a0.wait()
                s0 = pltpu.async_copy(rows0, acc_sh.at[rbuf.at[j]], ss0,
                                      add=True)
                a1.wait()
                s1 = pltpu.async_copy(rows1, acc_sh.at[rbuf.at[j + 1]], ss1,
                                      add=True)
                s0.wait()
                s1.wait()

        plsc.subcore_barrier()
        pltpu.sync_copy(acc_sh.at[pl.ds(base, RPT)],
                        out_hbm.at[pl.ds(c * NPAD + base, RPT)])

    return agg


_agg_feat = _make_agg(D)


def _prep_body(x_ref, da_ref, db_ref, dis_ref, g0_ref):
    deg = da_ref[:, :1] + db_ref[:, :1] + 1.0
    disb = jnp.broadcast_to(lax.rsqrt(deg), x_ref.shape)
    dis_ref[...] = disb
    g0_ref[...] = disb * x_ref[...]


def _prep(xp, deg2):
    return pl.pallas_call(
        _prep_body,
        grid=(NRB,),
        in_specs=[
            pl.BlockSpec((RBLK, D), lambda i: (i, 0)),
            pl.BlockSpec((RBLK, D), lambda i: (i, 0)),
            pl.BlockSpec((RBLK, D), lambda i: (i + NRB, 0)),
        ],
        out_specs=[
            pl.BlockSpec((RBLK, D), lambda i: (i, 0)),
            pl.BlockSpec((RBLK, D), lambda i: (i, 0)),
        ],
        out_shape=[
            jax.ShapeDtypeStruct((NPAD, D), jnp.float32),
            jax.ShapeDtypeStruct((NPAD, D), jnp.float32),
        ],
    )(xp, deg2, deg2)


def _layer_body(final, aa_ref, ab_ref, g_ref, dis_ref, w_ref, b_ref, o_ref):
    t = dis_ref[...] * (aa_ref[...] + ab_ref[...] + g_ref[...])
    h = jnp.dot(t, w_ref[...], preferred_element_type=jnp.float32) + b_ref[...]
    if final:
        o_ref[...] = h
    else:
        o_ref[...] = dis_ref[...] * jnp.maximum(h, 0.0)


def _layer(a2, g, dis, w, b, final):
    return pl.pallas_call(
        functools.partial(_layer_body, final),
        grid=(NRB,),
        in_specs=[
            pl.BlockSpec((RBLK, D), lambda i: (i, 0)),
            pl.BlockSpec((RBLK, D), lambda i: (i + NRB, 0)),
            pl.BlockSpec((RBLK, D), lambda i: (i, 0)),
            pl.BlockSpec((RBLK, D), lambda i: (i, 0)),
            pl.BlockSpec((D, D), lambda i: (0, 0)),
            pl.BlockSpec((1, D), lambda i: (0, 0)),
        ],
        out_specs=pl.BlockSpec((RBLK, D), lambda i: (i, 0)),
        out_shape=jax.ShapeDtypeStruct((NPAD, D), jnp.float32),
    )(a2, a2, g, dis, w, b)


def kernel(x, edge_index, W1, b1, W2, b2, W3, b3):
    xp = jnp.pad(x, ((0, NPAD - N), (0, 0)))
    row = edge_index[0].astype(jnp.int32)
    col = edge_index[1].astype(jnp.int32)
    pad = SLOTS - E
    pr = N + jnp.arange(pad, dtype=jnp.int32) % (NPAD - N)
    pc = jnp.arange(pad, dtype=jnp.int32) % N
    ridx = jnp.concatenate([row, pr]).reshape(NT * NBLK, B)
    cidx = jnp.concatenate([col, pc]).reshape(NT * NBLK, B)
    zeros_d = jnp.zeros((RPT, D), jnp.float32)
    ones_d = jnp.ones((NPAD, D), jnp.float32)

    deg2 = _agg_feat(ones_d, cidx, ridx, zeros_d)
    dis, g0 = _prep(xp, deg2)

    a1 = _agg_feat(g0, cidx, ridx, zeros_d)
    g1 = _layer(a1, g0, dis, W1, b1.reshape(1, D), final=False)
    a2 = _agg_feat(g1, cidx, ridx, zeros_d)
    g2 = _layer(a2, g1, dis, W2, b2.reshape(1, D), final=False)
    a3 = _agg_feat(g2, cidx, ridx, zeros_d)
    out = _layer(a3, g2, dis, W3, b3.reshape(1, D), final=True)
    return out[:N]


# speedup vs baseline: 18.1029x; 1.1158x over previous
import functools

import jax
import jax.numpy as jnp
from jax import lax
from jax.experimental import pallas as pl
from jax.experimental.pallas import tpu as pltpu
from jax.experimental.pallas import tpu_sc as plsc

N = 10000
D = 128
E = 320000
NPAD = 10240
NT = 32
B = 128
NBLK = 80
SLOTS = NT * NBLK * B
RPT = NPAD // 16
CB = 16
NCHUNK = NBLK // CB
RBLK = 256
NRB = NPAD // RBLK


def _make_agg(d):
    mesh = plsc.VectorSubcoreMesh(core_axis_name="c", subcore_axis_name="s")

    @functools.partial(
        pl.kernel,
        out_type=jax.ShapeDtypeStruct((2 * NPAD, d), jnp.float32),
        mesh=mesh,
        scratch_types=[
            pltpu.VMEM((CB, B), jnp.int32),
            pltpu.VMEM((CB, B), jnp.int32),
            pltpu.VMEM((B, d), jnp.float32),
            pltpu.VMEM((B, d), jnp.float32),
            pltpu.MemorySpace.VMEM_SHARED((NPAD, d), jnp.float32),
            pltpu.SemaphoreType.DMA,
            pltpu.SemaphoreType.DMA,
            pltpu.SemaphoreType.DMA,
            pltpu.SemaphoreType.DMA,
        ],
    )
    def agg(g_hbm, cidx_hbm, ridx_hbm, zero_hbm, out_hbm,
            cbuf, rbuf, rows0, rows1, acc_sh, gs0, gs1, ss0, ss1):
        c = lax.axis_index("c")
        s = lax.axis_index("s")
        wid = c * 16 + s
        base = s * RPT
        pltpu.sync_copy(zero_hbm, acc_sh.at[pl.ds(base, RPT)])
        plsc.subcore_barrier()

        @pl.loop(0, NCHUNK)
        def _(q):
            off = (wid * NCHUNK + q) * CB
            pltpu.sync_copy(cidx_hbm.at[pl.ds(off, CB)], cbuf)
            pltpu.sync_copy(ridx_hbm.at[pl.ds(off, CB)], rbuf)

            @pl.loop(0, CB, step=2)
            def _(j):
                a0 = pltpu.async_copy(g_hbm.at[cbuf.at[j]], rows0, gs0)
                a1 = pltpu.async_copy(g_hbm.at[cbuf.at[j + 1]], rows1, gs1)
                a0.wait()
                s0 = pltpu.async_copy(rows0, acc_sh.at[rbuf.at[j]], ss0,
                                      add=True)
                a1.wait()
                s1 = pltpu.async_copy(rows1, acc_sh.at[rbuf.at[j + 1]], ss1,
                                      add=True)
                s0.wait()
                s1.wait()

        plsc.subcore_barrier()
        pltpu.sync_copy(acc_sh.at[pl.ds(base, RPT)],
                        out_hbm.at[pl.ds(c * NPAD + base, RPT)])

    return agg


_agg_feat = _make_agg(D)

DDEG = D


def _make_deg():
    mesh = plsc.VectorSubcoreMesh(core_axis_name="c", subcore_axis_name="s")

    @functools.partial(
        pl.kernel,
        out_type=jax.ShapeDtypeStruct((2 * NPAD, DDEG), jnp.float32),
        mesh=mesh,
        scratch_types=[
            pltpu.VMEM((CB, B), jnp.int32),
            pltpu.VMEM((B, DDEG), jnp.float32),
            pltpu.MemorySpace.VMEM_SHARED((NPAD, DDEG), jnp.float32),
        ],
    )
    def deg(ridx_hbm, zero_hbm, ones_hbm, out_hbm, rbuf, ones_v, acc):
        c = lax.axis_index("c")
        s = lax.axis_index("s")
        wid = c * 16 + s
        base = s * RPT
        pltpu.sync_copy(zero_hbm, acc.at[pl.ds(base, RPT)])
        pltpu.sync_copy(ones_hbm, ones_v)
        plsc.subcore_barrier()

        @pl.loop(0, NCHUNK)
        def _(q):
            pltpu.sync_copy(
                ridx_hbm.at[pl.ds((wid * NCHUNK + q) * CB, CB)], rbuf)

            @pl.loop(0, CB)
            def _(j):
                pltpu.sync_copy(ones_v, acc.at[rbuf.at[j]], add=True)

        plsc.subcore_barrier()
        pltpu.sync_copy(acc.at[pl.ds(base, RPT)],
                        out_hbm.at[pl.ds(c * NPAD + base, RPT)])

    return deg


_deg16 = _make_deg()


def _prep_body(x_ref, da_ref, db_ref, dis_ref, g0_ref):
    deg = da_ref[:, :1] + db_ref[:, :1] + 1.0
    dis = lax.rsqrt(deg)
    dis_ref[...] = dis
    g0_ref[...] = dis * x_ref[...]


def _prep(xp, deg2):
    return pl.pallas_call(
        _prep_body,
        grid=(NRB,),
        in_specs=[
            pl.BlockSpec((RBLK, D), lambda i: (i, 0)),
            pl.BlockSpec((RBLK, DDEG), lambda i: (i, 0)),
            pl.BlockSpec((RBLK, DDEG), lambda i: (i + NRB, 0)),
        ],
        out_specs=[
            pl.BlockSpec((RBLK, 1), lambda i: (i, 0)),
            pl.BlockSpec((RBLK, D), lambda i: (i, 0)),
        ],
        out_shape=[
            jax.ShapeDtypeStruct((NPAD, 1), jnp.float32),
            jax.ShapeDtypeStruct((NPAD, D), jnp.float32),
        ],
    )(xp, deg2, deg2)


def _layer_body(final, aa_ref, ab_ref, g_ref, dis_ref, w_ref, b_ref, o_ref):
    t = dis_ref[...] * (aa_ref[...] + ab_ref[...] + g_ref[...])
    h = jnp.dot(t, w_ref[...], preferred_element_type=jnp.float32) + b_ref[...]
    if final:
        o_ref[...] = h
    else:
        o_ref[...] = dis_ref[...] * jnp.maximum(h, 0.0)


def _layer(a2, g, dis, w, b, final):
    return pl.pallas_call(
        functools.partial(_layer_body, final),
        grid=(NRB,),
        in_specs=[
            pl.BlockSpec((RBLK, D), lambda i: (i, 0)),
            pl.BlockSpec((RBLK, D), lambda i: (i + NRB, 0)),
            pl.BlockSpec((RBLK, D), lambda i: (i, 0)),
            pl.BlockSpec((RBLK, 1), lambda i: (i, 0)),
            pl.BlockSpec((D, D), lambda i: (0, 0)),
            pl.BlockSpec((1, D), lambda i: (0, 0)),
        ],
        out_specs=pl.BlockSpec((RBLK, D), lambda i: (i, 0)),
        out_shape=jax.ShapeDtypeStruct((NPAD, D), jnp.float32),
    )(a2, a2, g, dis, w, b)


def kernel(x, edge_index, W1, b1, W2, b2, W3, b3):
    xp = jnp.pad(x, ((0, NPAD - N), (0, 0)))
    row = edge_index[0].astype(jnp.int32)
    col = edge_index[1].astype(jnp.int32)
    pad = SLOTS - E
    pr = N + jnp.arange(pad, dtype=jnp.int32) % (NPAD - N)
    pc = jnp.arange(pad, dtype=jnp.int32) % N
    ridx = jnp.concatenate([row, pr]).reshape(NT * NBLK, B)
    cidx = jnp.concatenate([col, pc]).reshape(NT * NBLK, B)
    zeros_d = jnp.zeros((RPT, D), jnp.float32)
    ones_b = jnp.ones((B, DDEG), jnp.float32)
    deg2 = _deg16(ridx, zeros_d, ones_b)
    dis, g0 = _prep(xp, deg2)

    a1 = _agg_feat(g0, cidx, ridx, zeros_d)
    g1 = _layer(a1, g0, dis, W1, b1.reshape(1, D), final=False)
    a2 = _agg_feat(g1, cidx, ridx, zeros_d)
    g2 = _layer(a2, g1, dis, W2, b2.reshape(1, D), final=False)
    a3 = _agg_feat(g2, cidx, ridx, zeros_d)
    out = _layer(a3, g2, dis, W3, b3.reshape(1, D), final=True)
    return out[:N]


# speedup vs baseline: 18.1570x; 1.0030x over previous
import functools

import jax
import jax.numpy as jnp
from jax import lax
from jax.experimental import pallas as pl
from jax.experimental.pallas import tpu as pltpu
from jax.experimental.pallas import tpu_sc as plsc

N = 10000
D = 128
E = 320000
NPAD = 10240
NT = 32
B = 128
NBLK = 80
SLOTS = NT * NBLK * B
RPT = NPAD // 16
CB = 16
NCHUNK = NBLK // CB
RBLK = 256
NRB = NPAD // RBLK


def _make_agg(d):
    mesh = plsc.VectorSubcoreMesh(core_axis_name="c", subcore_axis_name="s")

    @functools.partial(
        pl.kernel,
        out_type=jax.ShapeDtypeStruct((2 * NPAD, d), jnp.float32),
        mesh=mesh,
        scratch_types=[
            pltpu.VMEM((CB, B), jnp.int32),
            pltpu.VMEM((CB, B), jnp.int32),
            pltpu.VMEM((B, d), jnp.float32),
            pltpu.VMEM((B, d), jnp.float32),
            pltpu.MemorySpace.VMEM_SHARED((NPAD, d), jnp.float32),
            pltpu.SemaphoreType.DMA,
            pltpu.SemaphoreType.DMA,
            pltpu.SemaphoreType.DMA,
            pltpu.SemaphoreType.DMA,
        ],
    )
    def agg(g_hbm, cidx_hbm, ridx_hbm, zero_hbm, out_hbm,
            cbuf, rbuf, rows0, rows1, acc_sh, gs0, gs1, ss0, ss1):
        c = lax.axis_index("c")
        s = lax.axis_index("s")
        wid = c * 16 + s
        base = s * RPT
        pltpu.sync_copy(zero_hbm, acc_sh.at[pl.ds(base, RPT)])
        plsc.subcore_barrier()

        @pl.loop(0, NCHUNK)
        def _(q):
            off = (wid * NCHUNK + q) * CB
            pltpu.sync_copy(cidx_hbm.at[pl.ds(off, CB)], cbuf)
            pltpu.sync_copy(ridx_hbm.at[pl.ds(off, CB)], rbuf)

            @pl.loop(0, CB, step=2)
            def _(j):
                a0 = pltpu.async_copy(g_hbm.at[cbuf.at[j]], rows0, gs0)
                a1 = pltpu.async_copy(g_hbm.at[cbuf.at[j + 1]], rows1, gs1)
                a0.wait()
                s0 = pltpu.async_copy(rows0, acc_sh.at[rbuf.at[j]], ss0,
                                      add=True)
                a1.wait()
                s1 = pltpu.async_copy(rows1, acc_sh.at[rbuf.at[j + 1]], ss1,
                                      add=True)
                s0.wait()
                s1.wait()

        plsc.subcore_barrier()
        pltpu.sync_copy(acc_sh.at[pl.ds(base, RPT)],
                        out_hbm.at[pl.ds(c * NPAD + base, RPT)])

    return agg


_agg_feat = _make_agg(D)

DDEG = D


def _make_deg():
    mesh = plsc.VectorSubcoreMesh(core_axis_name="c", subcore_axis_name="s")

    @functools.partial(
        pl.kernel,
        out_type=jax.ShapeDtypeStruct((2 * NPAD, DDEG), jnp.float32),
        mesh=mesh,
        scratch_types=[
            pltpu.VMEM((CB, B), jnp.int32),
            pltpu.VMEM((B, DDEG), jnp.float32),
            pltpu.MemorySpace.VMEM_SHARED((NPAD, DDEG), jnp.float32),
        ],
    )
    def deg(ridx_hbm, zero_hbm, ones_hbm, out_hbm, rbuf, ones_v, acc):
        c = lax.axis_index("c")
        s = lax.axis_index("s")
        wid = c * 16 + s
        base = s * RPT
        pltpu.sync_copy(zero_hbm, acc.at[pl.ds(base, RPT)])
        pltpu.sync_copy(ones_hbm, ones_v)
        plsc.subcore_barrier()

        @pl.loop(0, NCHUNK)
        def _(q):
            pltpu.sync_copy(
                ridx_hbm.at[pl.ds((wid * NCHUNK + q) * CB, CB)], rbuf)

            @pl.loop(0, CB)
            def _(j):
                pltpu.sync_copy(ones_v, acc.at[rbuf.at[j]], add=True)

        plsc.subcore_barrier()
        pltpu.sync_copy(acc.at[pl.ds(base, RPT)],
                        out_hbm.at[pl.ds(c * NPAD + base, RPT)])

    return deg


_deg16 = _make_deg()


def _prep_body(x_ref, da_ref, db_ref, dis_ref, g0_ref):
    deg = da_ref[:, :1] + db_ref[:, :1] + 1.0
    dis = lax.rsqrt(deg)
    dis_ref[...] = dis
    g0_ref[...] = dis * x_ref[...]


def _prep(xp, deg2):
    return pl.pallas_call(
        _prep_body,
        grid=(NRB,),
        in_specs=[
            pl.BlockSpec((RBLK, D), lambda i: (i, 0)),
            pl.BlockSpec((RBLK, DDEG), lambda i: (i, 0)),
            pl.BlockSpec((RBLK, DDEG), lambda i: (i + NRB, 0)),
        ],
        out_specs=[
            pl.BlockSpec((RBLK, 1), lambda i: (i, 0)),
            pl.BlockSpec((RBLK, D), lambda i: (i, 0)),
        ],
        out_shape=[
            jax.ShapeDtypeStruct((NPAD, 1), jnp.float32),
            jax.ShapeDtypeStruct((NPAD, D), jnp.float32),
        ],
    )(xp, deg2, deg2)


def _mm_body(x_ref, w_ref, o_ref):
    o_ref[...] = jnp.dot(x_ref[...], w_ref[...],
                         preferred_element_type=jnp.float32)


def _mm(x, w):
    return pl.pallas_call(
        _mm_body,
        grid=(NRB,),
        in_specs=[
            pl.BlockSpec((RBLK, D), lambda i: (i, 0)),
            pl.BlockSpec((D, D), lambda i: (0, 0)),
        ],
        out_specs=pl.BlockSpec((RBLK, D), lambda i: (i, 0)),
        out_shape=jax.ShapeDtypeStruct((NPAD, D), jnp.float32),
    )(x, w)


def _layer_body(final, aa_ref, ab_ref, g_ref, dis_ref, w_ref, b_ref, o_ref):
    t = dis_ref[...] * (aa_ref[...] + ab_ref[...] + g_ref[...]) + b_ref[...]
    if final:
        o_ref[...] = t
    else:
        h = jnp.maximum(t, 0.0)
        o_ref[...] = dis_ref[...] * jnp.dot(
            h, w_ref[...], preferred_element_type=jnp.float32)


def _layer(a2, g, dis, w_next, b, final):
    return pl.pallas_call(
        functools.partial(_layer_body, final),
        grid=(NRB,),
        in_specs=[
            pl.BlockSpec((RBLK, D), lambda i: (i, 0)),
            pl.BlockSpec((RBLK, D), lambda i: (i + NRB, 0)),
            pl.BlockSpec((RBLK, D), lambda i: (i, 0)),
            pl.BlockSpec((RBLK, 1), lambda i: (i, 0)),
            pl.BlockSpec((D, D), lambda i: (0, 0)),
            pl.BlockSpec((1, D), lambda i: (0, 0)),
        ],
        out_specs=pl.BlockSpec((RBLK, D), lambda i: (i, 0)),
        out_shape=jax.ShapeDtypeStruct((NPAD, D), jnp.float32),
    )(a2, a2, g, dis, w_next, b)


def kernel(x, edge_index, W1, b1, W2, b2, W3, b3):
    xp = jnp.pad(x, ((0, NPAD - N), (0, 0)))
    row = edge_index[0].astype(jnp.int32)
    col = edge_index[1].astype(jnp.int32)
    pad = SLOTS - E
    pr = N + jnp.arange(pad, dtype=jnp.int32) % (NPAD - N)
    pc = jnp.arange(pad, dtype=jnp.int32) % N
    ridx = jnp.concatenate([row, pr]).reshape(NT * NBLK, B)
    cidx = jnp.concatenate([col, pc]).reshape(NT * NBLK, B)
    zeros_d = jnp.zeros((RPT, D), jnp.float32)
    ones_b = jnp.ones((B, DDEG), jnp.float32)
    y = _mm(xp, W1)
    deg2 = _deg16(ridx, zeros_d, ones_b)
    dis, g0 = _prep(y, deg2)

    a1 = _agg_feat(g0, cidx, ridx, zeros_d)
    g1 = _layer(a1, g0, dis, W2, b1.reshape(1, D), final=False)
    a2 = _agg_feat(g1, cidx, ridx, zeros_d)
    g2 = _layer(a2, g1, dis, W3, b2.reshape(1, D), final=False)
    a3 = _agg_feat(g2, cidx, ridx, zeros_d)
    out = _layer(a3, g2, dis, W3, b3.reshape(1, D), final=True)
    return out[:N]


# speedup vs baseline: 18.3518x; 1.0107x over previous
import functools

import jax
import jax.numpy as jnp
from jax import lax
from jax.experimental import pallas as pl
from jax.experimental.pallas import tpu as pltpu
from jax.experimental.pallas import tpu_sc as plsc

N = 10000
D = 128
E = 320000
NPAD = 10240
NT = 32
B = 128
NBLK = 80
SLOTS = NT * NBLK * B
RPT = NPAD // 16
CB = 16
NCHUNK = NBLK // CB
RBLK = 256
NRB = NPAD // RBLK


def _make_agg(d):
    mesh = plsc.VectorSubcoreMesh(core_axis_name="c", subcore_axis_name="s")

    @functools.partial(
        pl.kernel,
        out_type=jax.ShapeDtypeStruct((2 * NPAD, d), jnp.float32),
        mesh=mesh,
        scratch_types=[
            pltpu.VMEM((CB, B), jnp.int32),
            pltpu.VMEM((CB, B), jnp.int32),
            pltpu.VMEM((B, d), jnp.float32),
            pltpu.VMEM((B, d), jnp.float32),
            pltpu.MemorySpace.VMEM_SHARED((NPAD, d), jnp.float32),
            pltpu.SemaphoreType.DMA,
            pltpu.SemaphoreType.DMA,
            pltpu.SemaphoreType.DMA,
            pltpu.SemaphoreType.DMA,
        ],
    )
    def agg(g_hbm, cidx_hbm, ridx_hbm, zero_hbm, out_hbm,
            cbuf, rbuf, rows0, rows1, acc_sh, gs0, gs1, ss0, ss1):
        c = lax.axis_index("c")
        s = lax.axis_index("s")
        wid = c * 16 + s
        base = s * RPT
        pltpu.sync_copy(zero_hbm, acc_sh.at[pl.ds(base, RPT)])
        plsc.subcore_barrier()

        @pl.loop(0, NCHUNK)
        def _(q):
            off = (wid * NCHUNK + q) * CB
            pltpu.sync_copy(cidx_hbm.at[pl.ds(off, CB)], cbuf)
            pltpu.sync_copy(ridx_hbm.at[pl.ds(off, CB)], rbuf)
            pltpu.async_copy(g_hbm.at[cbuf.at[0]], rows0, gs0)
            pltpu.async_copy(g_hbm.at[cbuf.at[1]], rows1, gs1)

            @pl.loop(0, CB, step=2)
            def _(j):
                pltpu.make_async_copy(g_hbm.at[cbuf.at[j]], rows0, gs0).wait()
                s0 = pltpu.async_copy(rows0, acc_sh.at[rbuf.at[j]], ss0,
                                      add=True)
                pltpu.make_async_copy(g_hbm.at[cbuf.at[j + 1]], rows1,
                                      gs1).wait()
                s1 = pltpu.async_copy(rows1, acc_sh.at[rbuf.at[j + 1]], ss1,
                                      add=True)
                s0.wait()

                @pl.when(j + 2 < CB)
                def _():
                    pltpu.async_copy(g_hbm.at[cbuf.at[j + 2]], rows0, gs0)

                s1.wait()

                @pl.when(j + 2 < CB)
                def _():
                    pltpu.async_copy(g_hbm.at[cbuf.at[j + 3]], rows1, gs1)

        plsc.subcore_barrier()
        pltpu.sync_copy(acc_sh.at[pl.ds(base, RPT)],
                        out_hbm.at[pl.ds(c * NPAD + base, RPT)])

    return agg


_agg_feat = _make_agg(D)

DDEG = D


def _make_deg():
    mesh = plsc.VectorSubcoreMesh(core_axis_name="c", subcore_axis_name="s")

    @functools.partial(
        pl.kernel,
        out_type=jax.ShapeDtypeStruct((2 * NPAD, DDEG), jnp.float32),
        mesh=mesh,
        scratch_types=[
            pltpu.VMEM((CB, B), jnp.int32),
            pltpu.VMEM((B, DDEG), jnp.float32),
            pltpu.MemorySpace.VMEM_SHARED((NPAD, DDEG), jnp.float32),
        ],
    )
    def deg(ridx_hbm, zero_hbm, ones_hbm, out_hbm, rbuf, ones_v, acc):
        c = lax.axis_index("c")
        s = lax.axis_index("s")
        wid = c * 16 + s
        base = s * RPT
        pltpu.sync_copy(zero_hbm, acc.at[pl.ds(base, RPT)])
        pltpu.sync_copy(ones_hbm, ones_v)
        plsc.subcore_barrier()

        @pl.loop(0, NCHUNK)
        def _(q):
            pltpu.sync_copy(
                ridx_hbm.at[pl.ds((wid * NCHUNK + q) * CB, CB)], rbuf)

            @pl.loop(0, CB)
            def _(j):
                pltpu.sync_copy(ones_v, acc.at[rbuf.at[j]], add=True)

        plsc.subcore_barrier()
        pltpu.sync_copy(acc.at[pl.ds(base, RPT)],
                        out_hbm.at[pl.ds(c * NPAD + base, RPT)])

    return deg


_deg16 = _make_deg()


def _prep_body(x_ref, da_ref, db_ref, dis_ref, g0_ref):
    deg = da_ref[:, :1] + db_ref[:, :1] + 1.0
    dis = lax.rsqrt(deg)
    dis_ref[...] = dis
    g0_ref[...] = dis * x_ref[...]


def _prep(xp, deg2):
    return pl.pallas_call(
        _prep_body,
        grid=(NRB,),
        in_specs=[
            pl.BlockSpec((RBLK, D), lambda i: (i, 0)),
            pl.BlockSpec((RBLK, DDEG), lambda i: (i, 0)),
            pl.BlockSpec((RBLK, DDEG), lambda i: (i + NRB, 0)),
        ],
        out_specs=[
            pl.BlockSpec((RBLK, 1), lambda i: (i, 0)),
            pl.BlockSpec((RBLK, D), lambda i: (i, 0)),
        ],
        out_shape=[
            jax.ShapeDtypeStruct((NPAD, 1), jnp.float32),
            jax.ShapeDtypeStruct((NPAD, D), jnp.float32),
        ],
    )(xp, deg2, deg2)


def _mm_body(x_ref, w_ref, o_ref):
    o_ref[...] = jnp.dot(x_ref[...], w_ref[...],
                         preferred_element_type=jnp.float32)


def _mm(x, w):
    return pl.pallas_call(
        _mm_body,
        grid=(NRB,),
        in_specs=[
            pl.BlockSpec((RBLK, D), lambda i: (i, 0)),
            pl.BlockSpec((D, D), lambda i: (0, 0)),
        ],
        out_specs=pl.BlockSpec((RBLK, D), lambda i: (i, 0)),
        out_shape=jax.ShapeDtypeStruct((NPAD, D), jnp.float32),
    )(x, w)


def _layer_body(final, aa_ref, ab_ref, g_ref, dis_ref, w_ref, b_ref, o_ref):
    t = dis_ref[...] * (aa_ref[...] + ab_ref[...] + g_ref[...]) + b_ref[...]
    if final:
        o_ref[...] = t
    else:
        h = jnp.maximum(t, 0.0)
        o_ref[...] = dis_ref[...] * jnp.dot(
            h, w_ref[...], preferred_element_type=jnp.float32)


def _layer(a2, g, dis, w_next, b, final):
    return pl.pallas_call(
        functools.partial(_layer_body, final),
        grid=(NRB,),
        in_specs=[
            pl.BlockSpec((RBLK, D), lambda i: (i, 0)),
            pl.BlockSpec((RBLK, D), lambda i: (i + NRB, 0)),
            pl.BlockSpec((RBLK, D), lambda i: (i, 0)),
            pl.BlockSpec((RBLK, 1), lambda i: (i, 0)),
            pl.BlockSpec((D, D), lambda i: (0, 0)),
            pl.BlockSpec((1, D), lambda i: (0, 0)),
        ],
        out_specs=pl.BlockSpec((RBLK, D), lambda i: (i, 0)),
        out_shape=jax.ShapeDtypeStruct((NPAD, D), jnp.float32),
    )(a2, a2, g, dis, w_next, b)


def kernel(x, edge_index, W1, b1, W2, b2, W3, b3):
    xp = jnp.pad(x, ((0, NPAD - N), (0, 0)))
    row = edge_index[0].astype(jnp.int32)
    col = edge_index[1].astype(jnp.int32)
    pad = SLOTS - E
    pr = N + jnp.arange(pad, dtype=jnp.int32) % (NPAD - N)
    pc = jnp.arange(pad, dtype=jnp.int32) % N
    ridx = jnp.concatenate([row, pr]).reshape(NT * NBLK, B)
    cidx = jnp.concatenate([col, pc]).reshape(NT * NBLK, B)
    zeros_d = jnp.zeros((RPT, D), jnp.float32)
    ones_b = jnp.ones((B, DDEG), jnp.float32)
    y = _mm(xp, W1)
    deg2 = _deg16(ridx, zeros_d, ones_b)
    dis, g0 = _prep(y, deg2)

    a1 = _agg_feat(g0, cidx, ridx, zeros_d)
    g1 = _layer(a1, g0, dis, W2, b1.reshape(1, D), final=False)
    a2 = _agg_feat(g1, cidx, ridx, zeros_d)
    g2 = _layer(a2, g1, dis, W3, b2.reshape(1, D), final=False)
    a3 = _agg_feat(g2, cidx, ridx, zeros_d)
    out = _layer(a3, g2, dis, W3, b3.reshape(1, D), final=True)
    return out[:N]


# speedup vs baseline: 18.3993x; 1.0026x over previous
import functools

import jax
import jax.numpy as jnp
from jax import lax
from jax.experimental import pallas as pl
from jax.experimental.pallas import tpu as pltpu
from jax.experimental.pallas import tpu_sc as plsc

N = 10000
D = 128
E = 320000
NPAD = 10240
NT = 32
B = 128
NBLK = 80
SLOTS = NT * NBLK * B
RPT = NPAD // 16
CB = 16
NCHUNK = NBLK // CB
RBLK = 256
NRB = NPAD // RBLK


def _make_agg(d):
    mesh = plsc.VectorSubcoreMesh(core_axis_name="c", subcore_axis_name="s")

    @functools.partial(
        pl.kernel,
        out_type=jax.ShapeDtypeStruct((2 * NPAD, d), jnp.float32),
        mesh=mesh,
        scratch_types=[
            pltpu.VMEM((CB, B), jnp.int32),
            pltpu.VMEM((CB, B), jnp.int32),
            pltpu.VMEM((B, d), jnp.float32),
            pltpu.VMEM((B, d), jnp.float32),
            pltpu.MemorySpace.VMEM_SHARED((NPAD, d), jnp.float32),
            pltpu.SemaphoreType.DMA,
            pltpu.SemaphoreType.DMA,
            pltpu.SemaphoreType.DMA,
            pltpu.SemaphoreType.DMA,
        ],
    )
    def agg(g_hbm, cidx_hbm, ridx_hbm, zero_hbm, out_hbm,
            cbuf, rbuf, rows0, rows1, acc_sh, gs0, gs1, ss0, ss1):
        c = lax.axis_index("c")
        s = lax.axis_index("s")
        wid = c * 16 + s
        base = s * RPT
        pltpu.sync_copy(zero_hbm, acc_sh.at[pl.ds(base, RPT)])
        plsc.subcore_barrier()

        @pl.loop(0, NCHUNK)
        def _(q):
            off = (wid * NCHUNK + q) * CB
            pltpu.sync_copy(cidx_hbm.at[pl.ds(off, CB)], cbuf)
            pltpu.sync_copy(ridx_hbm.at[pl.ds(off, CB)], rbuf)
            pltpu.async_copy(g_hbm.at[cbuf.at[0]], rows0, gs0)
            pltpu.async_copy(g_hbm.at[cbuf.at[1]], rows1, gs1)

            @pl.loop(0, CB, step=2)
            def _(j):
                pltpu.make_async_copy(g_hbm.at[cbuf.at[j]], rows0, gs0).wait()
                s0 = pltpu.async_copy(rows0, acc_sh.at[rbuf.at[j]], ss0,
                                      add=True)
                pltpu.make_async_copy(g_hbm.at[cbuf.at[j + 1]], rows1,
                                      gs1).wait()
                s1 = pltpu.async_copy(rows1, acc_sh.at[rbuf.at[j + 1]], ss1,
                                      add=True)
                s0.wait()

                @pl.when(j + 2 < CB)
                def _():
                    pltpu.async_copy(g_hbm.at[cbuf.at[j + 2]], rows0, gs0)

                s1.wait()

                @pl.when(j + 2 < CB)
                def _():
                    pltpu.async_copy(g_hbm.at[cbuf.at[j + 3]], rows1, gs1)

        plsc.subcore_barrier()
        pltpu.sync_copy(acc_sh.at[pl.ds(base, RPT)],
                        out_hbm.at[pl.ds(c * NPAD + base, RPT)])

    return agg


_agg_feat = _make_agg(D)

DDEG = D


def _make_deg():
    mesh = plsc.VectorSubcoreMesh(core_axis_name="c", subcore_axis_name="s")

    @functools.partial(
        pl.kernel,
        out_type=jax.ShapeDtypeStruct((2 * NPAD, DDEG), jnp.float32),
        mesh=mesh,
        scratch_types=[
            pltpu.VMEM((CB, B), jnp.int32),
            pltpu.VMEM((B, DDEG), jnp.float32),
            pltpu.MemorySpace.VMEM_SHARED((NPAD, DDEG), jnp.float32),
        ],
    )
    def deg(ridx_hbm, zero_hbm, ones_hbm, out_hbm, rbuf, ones_v, acc):
        c = lax.axis_index("c")
        s = lax.axis_index("s")
        wid = c * 16 + s
        base = s * RPT
        pltpu.sync_copy(zero_hbm, acc.at[pl.ds(base, RPT)])
        pltpu.sync_copy(ones_hbm, ones_v)
        plsc.subcore_barrier()

        @pl.loop(0, NCHUNK)
        def _(q):
            pltpu.sync_copy(
                ridx_hbm.at[pl.ds((wid * NCHUNK + q) * CB, CB)], rbuf)

            @pl.loop(0, CB)
            def _(j):
                pltpu.sync_copy(ones_v, acc.at[rbuf.at[j]], add=True)

        plsc.subcore_barrier()
        pltpu.sync_copy(acc.at[pl.ds(base, RPT)],
                        out_hbm.at[pl.ds(c * NPAD + base, RPT)])

    return deg


_deg16 = _make_deg()


def _prep_body(x_ref, w_ref, da_ref, db_ref, dis_ref, g0_ref):
    deg = da_ref[:, :1] + db_ref[:, :1] + 1.0
    dis = lax.rsqrt(deg)
    dis_ref[...] = dis
    g0_ref[...] = dis * jnp.dot(x_ref[...], w_ref[...],
                                preferred_element_type=jnp.float32)


def _prep(xp, w1, deg2):
    return pl.pallas_call(
        _prep_body,
        grid=(NRB,),
        in_specs=[
            pl.BlockSpec((RBLK, D), lambda i: (i, 0)),
            pl.BlockSpec((D, D), lambda i: (0, 0)),
            pl.BlockSpec((RBLK, DDEG), lambda i: (i, 0)),
            pl.BlockSpec((RBLK, DDEG), lambda i: (i + NRB, 0)),
        ],
        out_specs=[
            pl.BlockSpec((RBLK, 1), lambda i: (i, 0)),
            pl.BlockSpec((RBLK, D), lambda i: (i, 0)),
        ],
        out_shape=[
            jax.ShapeDtypeStruct((NPAD, 1), jnp.float32),
            jax.ShapeDtypeStruct((NPAD, D), jnp.float32),
        ],
    )(xp, w1, deg2, deg2)


def _layer_body(final, aa_ref, ab_ref, g_ref, dis_ref, w_ref, b_ref, o_ref):
    t = dis_ref[...] * (aa_ref[...] + ab_ref[...] + g_ref[...]) + b_ref[...]
    if final:
        o_ref[...] = t
    else:
        h = jnp.maximum(t, 0.0)
        o_ref[...] = dis_ref[...] * jnp.dot(
            h, w_ref[...], preferred_element_type=jnp.float32)


def _layer(a2, g, dis, w_next, b, final):
    nrows = N if final else NPAD
    return pl.pallas_call(
        functools.partial(_layer_body, final),
        grid=(NRB,),
        in_specs=[
            pl.BlockSpec((RBLK, D), lambda i: (i, 0)),
            pl.BlockSpec((RBLK, D), lambda i: (i + NRB, 0)),
            pl.BlockSpec((RBLK, D), lambda i: (i, 0)),
            pl.BlockSpec((RBLK, 1), lambda i: (i, 0)),
            pl.BlockSpec((D, D), lambda i: (0, 0)),
            pl.BlockSpec((1, D), lambda i: (0, 0)),
        ],
        out_specs=pl.BlockSpec((RBLK, D), lambda i: (i, 0)),
        out_shape=jax.ShapeDtypeStruct((nrows, D), jnp.float32),
    )(a2, a2, g, dis, w_next, b)


def kernel(x, edge_index, W1, b1, W2, b2, W3, b3):
    xp = jnp.pad(x, ((0, NPAD - N), (0, 0)))
    row = edge_index[0].astype(jnp.int32)
    col = edge_index[1].astype(jnp.int32)
    pad = SLOTS - E
    pr = N + jnp.arange(pad, dtype=jnp.int32) % (NPAD - N)
    pc = jnp.arange(pad, dtype=jnp.int32) % N
    ridx = jnp.concatenate([row, pr]).reshape(NT * NBLK, B)
    cidx = jnp.concatenate([col, pc]).reshape(NT * NBLK, B)
    zeros_d = jnp.zeros((RPT, D), jnp.float32)
    ones_b = jnp.ones((B, DDEG), jnp.float32)
    deg2 = _deg16(ridx, zeros_d, ones_b)
    dis, g0 = _prep(xp, W1, deg2)

    a1 = _agg_feat(g0, cidx, ridx, zeros_d)
    g1 = _layer(a1, g0, dis, W2, b1.reshape(1, D), final=False)
    a2 = _agg_feat(g1, cidx, ridx, zeros_d)
    g2 = _layer(a2, g1, dis, W3, b2.reshape(1, D), final=False)
    a3 = _agg_feat(g2, cidx, ridx, zeros_d)
    return _layer(a3, g2, dis, W3, b3.reshape(1, D), final=True)


# speedup vs baseline: 18.9744x; 1.0313x over previous
import functools

import jax
import jax.numpy as jnp
from jax import lax
from jax.experimental import pallas as pl
from jax.experimental.pallas import tpu as pltpu
from jax.experimental.pallas import tpu_sc as plsc

N = 10000
D = 128
E = 320000
NPAD = 10240
NT = 32
B = 128
NBLK = 80
SLOTS = NT * NBLK * B
RPT = NPAD // 16
CB = 40
NCHUNK = NBLK // CB
RBLK = 256
NRB = NPAD // RBLK


def _make_agg(d):
    mesh = plsc.VectorSubcoreMesh(core_axis_name="c", subcore_axis_name="s")

    @functools.partial(
        pl.kernel,
        out_type=jax.ShapeDtypeStruct((2 * NPAD, d), jnp.float32),
        mesh=mesh,
        scratch_types=[
            pltpu.VMEM((CB, B), jnp.int32),
            pltpu.VMEM((CB, B), jnp.int32),
            pltpu.VMEM((B, d), jnp.float32),
            pltpu.VMEM((B, d), jnp.float32),
            pltpu.MemorySpace.VMEM_SHARED((NPAD, d), jnp.float32),
            pltpu.SemaphoreType.DMA,
            pltpu.SemaphoreType.DMA,
            pltpu.SemaphoreType.DMA,
            pltpu.SemaphoreType.DMA,
        ],
    )
    def agg(g_hbm, cidx_hbm, ridx_hbm, zero_hbm, out_hbm,
            cbuf, rbuf, rows0, rows1, acc_sh, gs0, gs1, ss0, ss1):
        c = lax.axis_index("c")
        s = lax.axis_index("s")
        wid = c * 16 + s
        base = s * RPT
        pltpu.sync_copy(cidx_hbm.at[pl.ds(wid * NBLK, CB)], cbuf)
        pltpu.sync_copy(ridx_hbm.at[pl.ds(wid * NBLK, CB)], rbuf)
        pltpu.async_copy(g_hbm.at[cbuf.at[0]], rows0, gs0)
        pltpu.async_copy(g_hbm.at[cbuf.at[1]], rows1, gs1)
        pltpu.sync_copy(zero_hbm, acc_sh.at[pl.ds(base, RPT)])
        plsc.subcore_barrier()

        for q in range(NCHUNK):
            if q > 0:
                off = wid * NBLK + q * CB
                pltpu.sync_copy(cidx_hbm.at[pl.ds(off, CB)], cbuf)
                pltpu.sync_copy(ridx_hbm.at[pl.ds(off, CB)], rbuf)
                pltpu.async_copy(g_hbm.at[cbuf.at[0]], rows0, gs0)
                pltpu.async_copy(g_hbm.at[cbuf.at[1]], rows1, gs1)

            @pl.loop(0, CB, step=2)
            def _(j):
                pltpu.make_async_copy(g_hbm.at[cbuf.at[j]], rows0, gs0).wait()
                s0 = pltpu.async_copy(rows0, acc_sh.at[rbuf.at[j]], ss0,
                                      add=True)
                pltpu.make_async_copy(g_hbm.at[cbuf.at[j + 1]], rows1,
                                      gs1).wait()
                s1 = pltpu.async_copy(rows1, acc_sh.at[rbuf.at[j + 1]], ss1,
                                      add=True)
                s0.wait()

                @pl.when(j + 2 < CB)
                def _():
                    pltpu.async_copy(g_hbm.at[cbuf.at[j + 2]], rows0, gs0)

                s1.wait()

                @pl.when(j + 2 < CB)
                def _():
                    pltpu.async_copy(g_hbm.at[cbuf.at[j + 3]], rows1, gs1)

        plsc.subcore_barrier()
        pltpu.sync_copy(acc_sh.at[pl.ds(base, RPT)],
                        out_hbm.at[pl.ds(c * NPAD + base, RPT)])

    return agg


_agg_feat = _make_agg(D)

DDEG = D


def _make_deg():
    mesh = plsc.VectorSubcoreMesh(core_axis_name="c", subcore_axis_name="s")

    @functools.partial(
        pl.kernel,
        out_type=jax.ShapeDtypeStruct((2 * NPAD, DDEG), jnp.float32),
        mesh=mesh,
        scratch_types=[
            pltpu.VMEM((CB, B), jnp.int32),
            pltpu.VMEM((B, DDEG), jnp.float32),
            pltpu.MemorySpace.VMEM_SHARED((NPAD, DDEG), jnp.float32),
        ],
    )
    def deg(ridx_hbm, zero_hbm, ones_hbm, out_hbm, rbuf, ones_v, acc):
        c = lax.axis_index("c")
        s = lax.axis_index("s")
        wid = c * 16 + s
        base = s * RPT
        pltpu.sync_copy(zero_hbm, acc.at[pl.ds(base, RPT)])
        pltpu.sync_copy(ones_hbm, ones_v)
        plsc.subcore_barrier()

        @pl.loop(0, NCHUNK)
        def _(q):
            pltpu.sync_copy(
                ridx_hbm.at[pl.ds((wid * NCHUNK + q) * CB, CB)], rbuf)

            @pl.loop(0, CB)
            def _(j):
                pltpu.sync_copy(ones_v, acc.at[rbuf.at[j]], add=True)

        plsc.subcore_barrier()
        pltpu.sync_copy(acc.at[pl.ds(base, RPT)],
                        out_hbm.at[pl.ds(c * NPAD + base, RPT)])

    return deg


_deg16 = _make_deg()


def _prep_body(x_ref, w_ref, da_ref, db_ref, dis_ref, g0_ref):
    deg = da_ref[:, :1] + db_ref[:, :1] + 1.0
    dis = lax.rsqrt(deg)
    dis_ref[...] = dis
    g0_ref[...] = dis * jnp.dot(x_ref[...], w_ref[...],
                                preferred_element_type=jnp.float32)


def _prep(xp, w1, deg2):
    return pl.pallas_call(
        _prep_body,
        grid=(NRB,),
        in_specs=[
            pl.BlockSpec((RBLK, D), lambda i: (i, 0)),
            pl.BlockSpec((D, D), lambda i: (0, 0)),
            pl.BlockSpec((RBLK, DDEG), lambda i: (i, 0)),
            pl.BlockSpec((RBLK, DDEG), lambda i: (i + NRB, 0)),
        ],
        out_specs=[
            pl.BlockSpec((RBLK, 1), lambda i: (i, 0)),
            pl.BlockSpec((RBLK, D), lambda i: (i, 0)),
        ],
        out_shape=[
            jax.ShapeDtypeStruct((NPAD, 1), jnp.float32),
            jax.ShapeDtypeStruct((NPAD, D), jnp.float32),
        ],
    )(xp, w1, deg2, deg2)


def _layer_body(final, aa_ref, ab_ref, g_ref, dis_ref, w_ref, b_ref, o_ref):
    t = dis_ref[...] * (aa_ref[...] + ab_ref[...] + g_ref[...]) + b_ref[...]
    if final:
        o_ref[...] = t
    else:
        h = jnp.maximum(t, 0.0)
        o_ref[...] = dis_ref[...] * jnp.dot(
            h, w_ref[...], preferred_element_type=jnp.float32)


def _layer(a2, g, dis, w_next, b, final):
    nrows = N if final else NPAD
    return pl.pallas_call(
        functools.partial(_layer_body, final),
        grid=(NRB,),
        in_specs=[
            pl.BlockSpec((RBLK, D), lambda i: (i, 0)),
            pl.BlockSpec((RBLK, D), lambda i: (i + NRB, 0)),
            pl.BlockSpec((RBLK, D), lambda i: (i, 0)),
            pl.BlockSpec((RBLK, 1), lambda i: (i, 0)),
            pl.BlockSpec((D, D), lambda i: (0, 0)),
            pl.BlockSpec((1, D), lambda i: (0, 0)),
        ],
        out_specs=pl.BlockSpec((RBLK, D), lambda i: (i, 0)),
        out_shape=jax.ShapeDtypeStruct((nrows, D), jnp.float32),
    )(a2, a2, g, dis, w_next, b)


def kernel(x, edge_index, W1, b1, W2, b2, W3, b3):
    xp = jnp.pad(x, ((0, NPAD - N), (0, 0)))
    row = edge_index[0].astype(jnp.int32)
    col = edge_index[1].astype(jnp.int32)
    pad = SLOTS - E
    pr = N + jnp.arange(pad, dtype=jnp.int32) % (NPAD - N)
    pc = jnp.arange(pad, dtype=jnp.int32) % N
    ridx = jnp.concatenate([row, pr]).reshape(NT * NBLK, B)
    cidx = jnp.concatenate([col, pc]).reshape(NT * NBLK, B)
    zeros_d = jnp.zeros((RPT, D), jnp.float32)
    ones_b = jnp.ones((B, DDEG), jnp.float32)
    deg2 = _deg16(ridx, zeros_d, ones_b)
    dis, g0 = _prep(xp, W1, deg2)

    a1 = _agg_feat(g0, cidx, ridx, zeros_d)
    g1 = _layer(a1, g0, dis, W2, b1.reshape(1, D), final=False)
    a2 = _agg_feat(g1, cidx, ridx, zeros_d)
    g2 = _layer(a2, g1, dis, W3, b2.reshape(1, D), final=False)
    a3 = _agg_feat(g2, cidx, ridx, zeros_d)
    return _layer(a3, g2, dis, W3, b3.reshape(1, D), final=True)


# speedup vs baseline: 20.1955x; 1.0644x over previous
import functools

import jax
import jax.numpy as jnp
from jax import lax
from jax.experimental import pallas as pl
from jax.experimental.pallas import tpu as pltpu
from jax.experimental.pallas import tpu_sc as plsc

N = 10000
D = 128
E = 320000
NPAD = 10240
NT = 32
B = 128
NBLK = 80
SLOTS = NT * NBLK * B
RPT = NPAD // 16
CB = 40
NCHUNK = NBLK // CB
RBLK = 512
NRB = NPAD // RBLK


def _make_agg(d):
    mesh = plsc.VectorSubcoreMesh(core_axis_name="c", subcore_axis_name="s")

    @functools.partial(
        pl.kernel,
        out_type=jax.ShapeDtypeStruct((2 * NPAD, d), jnp.float32),
        mesh=mesh,
        scratch_types=[
            pltpu.VMEM((CB, B), jnp.int32),
            pltpu.VMEM((CB, B), jnp.int32),
            pltpu.VMEM((B, d), jnp.float32),
            pltpu.VMEM((B, d), jnp.float32),
            pltpu.MemorySpace.VMEM_SHARED((NPAD, d), jnp.float32),
            pltpu.SemaphoreType.DMA,
            pltpu.SemaphoreType.DMA,
            pltpu.SemaphoreType.DMA,
            pltpu.SemaphoreType.DMA,
        ],
    )
    def agg(g_hbm, cidx_hbm, ridx_hbm, zero_hbm, out_hbm,
            cbuf, rbuf, rows0, rows1, acc_sh, gs0, gs1, ss0, ss1):
        c = lax.axis_index("c")
        s = lax.axis_index("s")
        wid = c * 16 + s
        base = s * RPT
        pltpu.sync_copy(cidx_hbm.at[pl.ds(wid * NBLK, CB)], cbuf)
        pltpu.sync_copy(ridx_hbm.at[pl.ds(wid * NBLK, CB)], rbuf)
        pltpu.async_copy(g_hbm.at[cbuf.at[0]], rows0, gs0)
        pltpu.async_copy(g_hbm.at[cbuf.at[1]], rows1, gs1)
        pltpu.sync_copy(zero_hbm, acc_sh.at[pl.ds(base, RPT)])
        plsc.subcore_barrier()

        for q in range(NCHUNK):
            if q > 0:
                off = wid * NBLK + q * CB
                pltpu.sync_copy(cidx_hbm.at[pl.ds(off, CB)], cbuf)
                pltpu.sync_copy(ridx_hbm.at[pl.ds(off, CB)], rbuf)
                pltpu.async_copy(g_hbm.at[cbuf.at[0]], rows0, gs0)
                pltpu.async_copy(g_hbm.at[cbuf.at[1]], rows1, gs1)

            @pl.loop(0, CB, step=2)
            def _(j):
                pltpu.make_async_copy(g_hbm.at[cbuf.at[j]], rows0, gs0).wait()
                s0 = pltpu.async_copy(rows0, acc_sh.at[rbuf.at[j]], ss0,
                                      add=True)
                pltpu.make_async_copy(g_hbm.at[cbuf.at[j + 1]], rows1,
                                      gs1).wait()
                s1 = pltpu.async_copy(rows1, acc_sh.at[rbuf.at[j + 1]], ss1,
                                      add=True)
                s0.wait()

                @pl.when(j + 2 < CB)
                def _():
                    pltpu.async_copy(g_hbm.at[cbuf.at[j + 2]], rows0, gs0)

                s1.wait()

                @pl.when(j + 2 < CB)
                def _():
                    pltpu.async_copy(g_hbm.at[cbuf.at[j + 3]], rows1, gs1)

        plsc.subcore_barrier()
        pltpu.sync_copy(acc_sh.at[pl.ds(base, RPT)],
                        out_hbm.at[pl.ds(c * NPAD + base, RPT)])

    return agg


_agg_feat = _make_agg(D)

DDEG = D


def _make_deg():
    mesh = plsc.VectorSubcoreMesh(core_axis_name="c", subcore_axis_name="s")

    @functools.partial(
        pl.kernel,
        out_type=jax.ShapeDtypeStruct((2 * NPAD, DDEG), jnp.float32),
        mesh=mesh,
        scratch_types=[
            pltpu.VMEM((CB, B), jnp.int32),
            pltpu.VMEM((B, DDEG), jnp.float32),
            pltpu.MemorySpace.VMEM_SHARED((NPAD, DDEG), jnp.float32),
        ],
    )
    def deg(ridx_hbm, zero_hbm, ones_hbm, out_hbm, rbuf, ones_v, acc):
        c = lax.axis_index("c")
        s = lax.axis_index("s")
        wid = c * 16 + s
        base = s * RPT
        pltpu.sync_copy(zero_hbm, acc.at[pl.ds(base, RPT)])
        pltpu.sync_copy(ones_hbm, ones_v)
        plsc.subcore_barrier()

        @pl.loop(0, NCHUNK)
        def _(q):
            pltpu.sync_copy(
                ridx_hbm.at[pl.ds((wid * NCHUNK + q) * CB, CB)], rbuf)

            @pl.loop(0, CB)
            def _(j):
                pltpu.sync_copy(ones_v, acc.at[rbuf.at[j]], add=True)

        plsc.subcore_barrier()
        pltpu.sync_copy(acc.at[pl.ds(base, RPT)],
                        out_hbm.at[pl.ds(c * NPAD + base, RPT)])

    return deg


_deg16 = _make_deg()


def _prep_body(x_ref, w_ref, da_ref, db_ref, dis_ref, g0_ref):
    deg = da_ref[:, :1] + db_ref[:, :1] + 1.0
    dis = lax.rsqrt(deg)
    dis_ref[...] = dis
    g0_ref[...] = dis * jnp.dot(x_ref[...], w_ref[...],
                                preferred_element_type=jnp.float32)


def _prep(xp, w1, deg2):
    return pl.pallas_call(
        _prep_body,
        grid=(NRB,),
        in_specs=[
            pl.BlockSpec((RBLK, D), lambda i: (i, 0)),
            pl.BlockSpec((D, D), lambda i: (0, 0)),
            pl.BlockSpec((RBLK, DDEG), lambda i: (i, 0)),
            pl.BlockSpec((RBLK, DDEG), lambda i: (i + NRB, 0)),
        ],
        out_specs=[
            pl.BlockSpec((RBLK, 1), lambda i: (i, 0)),
            pl.BlockSpec((RBLK, D), lambda i: (i, 0)),
        ],
        out_shape=[
            jax.ShapeDtypeStruct((NPAD, 1), jnp.float32),
            jax.ShapeDtypeStruct((NPAD, D), jnp.float32),
        ],
    )(xp, w1, deg2, deg2)


def _layer_body(final, aa_ref, ab_ref, g_ref, dis_ref, w_ref, b_ref, o_ref):
    t = dis_ref[...] * (aa_ref[...] + ab_ref[...] + g_ref[...]) + b_ref[...]
    if final:
        o_ref[...] = t
    else:
        h = jnp.maximum(t, 0.0)
        o_ref[...] = dis_ref[...] * jnp.dot(
            h, w_ref[...], preferred_element_type=jnp.float32)


def _layer(a2, g, dis, w_next, b, final):
    nrows = N if final else NPAD
    return pl.pallas_call(
        functools.partial(_layer_body, final),
        grid=(NRB,),
        in_specs=[
            pl.BlockSpec((RBLK, D), lambda i: (i, 0)),
            pl.BlockSpec((RBLK, D), lambda i: (i + NRB, 0)),
            pl.BlockSpec((RBLK, D), lambda i: (i, 0)),
            pl.BlockSpec((RBLK, 1), lambda i: (i, 0)),
            pl.BlockSpec((D, D), lambda i: (0, 0)),
            pl.BlockSpec((1, D), lambda i: (0, 0)),
        ],
        out_specs=pl.BlockSpec((RBLK, D), lambda i: (i, 0)),
        out_shape=jax.ShapeDtypeStruct((nrows, D), jnp.float32),
    )(a2, a2, g, dis, w_next, b)


def kernel(x, edge_index, W1, b1, W2, b2, W3, b3):
    xp = jnp.pad(x, ((0, NPAD - N), (0, 0)))
    row = edge_index[0].astype(jnp.int32)
    col = edge_index[1].astype(jnp.int32)
    pad = SLOTS - E
    pr = N + jnp.arange(pad, dtype=jnp.int32) % (NPAD - N)
    pc = jnp.arange(pad, dtype=jnp.int32) % N
    ridx = jnp.concatenate([row, pr]).reshape(NT * NBLK, B)
    cidx = jnp.concatenate([col, pc]).reshape(NT * NBLK, B)
    zeros_d = jnp.zeros((RPT, D), jnp.float32)
    ones_b = jnp.ones((B, DDEG), jnp.float32)
    deg2 = _deg16(ridx, zeros_d, ones_b)
    dis, g0 = _prep(xp, W1, deg2)

    a1 = _agg_feat(g0, cidx, ridx, zeros_d)
    g1 = _layer(a1, g0, dis, W2, b1.reshape(1, D), final=False)
    a2 = _agg_feat(g1, cidx, ridx, zeros_d)
    g2 = _layer(a2, g1, dis, W3, b2.reshape(1, D), final=False)
    a3 = _agg_feat(g2, cidx, ridx, zeros_d)
    return _layer(a3, g2, dis, W3, b3.reshape(1, D), final=True)


# speedup vs baseline: 20.9382x; 1.0368x over previous
import functools

import jax
import jax.numpy as jnp
from jax import lax
from jax.experimental import pallas as pl
from jax.experimental.pallas import tpu as pltpu
from jax.experimental.pallas import tpu_sc as plsc

N = 10000
D = 128
E = 320000
NPAD = 10240
NT = 32
B = 128
NBLK = 80
SLOTS = NT * NBLK * B
RPT = NPAD // 16
CB = 40
NCHUNK = NBLK // CB
RBLK = 1024
NRB = NPAD // RBLK


def _make_agg(d):
    mesh = plsc.VectorSubcoreMesh(core_axis_name="c", subcore_axis_name="s")

    @functools.partial(
        pl.kernel,
        out_type=jax.ShapeDtypeStruct((2 * NPAD, d), jnp.float32),
        mesh=mesh,
        scratch_types=[
            pltpu.VMEM((CB, B), jnp.int32),
            pltpu.VMEM((CB, B), jnp.int32),
            pltpu.VMEM((B, d), jnp.float32),
            pltpu.VMEM((B, d), jnp.float32),
            pltpu.MemorySpace.VMEM_SHARED((NPAD, d), jnp.float32),
            pltpu.SemaphoreType.DMA,
            pltpu.SemaphoreType.DMA,
            pltpu.SemaphoreType.DMA,
            pltpu.SemaphoreType.DMA,
        ],
    )
    def agg(g_hbm, cidx_hbm, ridx_hbm, zero_hbm, out_hbm,
            cbuf, rbuf, rows0, rows1, acc_sh, gs0, gs1, ss0, ss1):
        c = lax.axis_index("c")
        s = lax.axis_index("s")
        wid = c * 16 + s
        base = s * RPT
        pltpu.sync_copy(cidx_hbm.at[pl.ds(wid * NBLK, CB)], cbuf)
        pltpu.sync_copy(ridx_hbm.at[pl.ds(wid * NBLK, CB)], rbuf)
        pltpu.async_copy(g_hbm.at[cbuf.at[0]], rows0, gs0)
        pltpu.async_copy(g_hbm.at[cbuf.at[1]], rows1, gs1)
        pltpu.sync_copy(zero_hbm, acc_sh.at[pl.ds(base, RPT)])
        plsc.subcore_barrier()

        for q in range(NCHUNK):
            if q > 0:
                off = wid * NBLK + q * CB
                pltpu.sync_copy(cidx_hbm.at[pl.ds(off, CB)], cbuf)
                pltpu.sync_copy(ridx_hbm.at[pl.ds(off, CB)], rbuf)
                pltpu.async_copy(g_hbm.at[cbuf.at[0]], rows0, gs0)
                pltpu.async_copy(g_hbm.at[cbuf.at[1]], rows1, gs1)

            @pl.loop(0, CB, step=2)
            def _(j):
                pltpu.make_async_copy(g_hbm.at[cbuf.at[j]], rows0, gs0).wait()
                s0 = pltpu.async_copy(rows0, acc_sh.at[rbuf.at[j]], ss0,
                                      add=True)
                pltpu.make_async_copy(g_hbm.at[cbuf.at[j + 1]], rows1,
                                      gs1).wait()
                s1 = pltpu.async_copy(rows1, acc_sh.at[rbuf.at[j + 1]], ss1,
                                      add=True)
                s0.wait()

                @pl.when(j + 2 < CB)
                def _():
                    pltpu.async_copy(g_hbm.at[cbuf.at[j + 2]], rows0, gs0)

                s1.wait()

                @pl.when(j + 2 < CB)
                def _():
                    pltpu.async_copy(g_hbm.at[cbuf.at[j + 3]], rows1, gs1)

        plsc.subcore_barrier()
        pltpu.sync_copy(acc_sh.at[pl.ds(base, RPT)],
                        out_hbm.at[pl.ds(c * NPAD + base, RPT)])

    return agg


_agg_feat = _make_agg(D)

DDEG = D


def _make_deg():
    mesh = plsc.VectorSubcoreMesh(core_axis_name="c", subcore_axis_name="s")

    @functools.partial(
        pl.kernel,
        out_type=jax.ShapeDtypeStruct((2 * NPAD, DDEG), jnp.float32),
        mesh=mesh,
        scratch_types=[
            pltpu.VMEM((CB, B), jnp.int32),
            pltpu.VMEM((B, DDEG), jnp.float32),
            pltpu.MemorySpace.VMEM_SHARED((NPAD, DDEG), jnp.float32),
        ],
    )
    def deg(ridx_hbm, zero_hbm, ones_hbm, out_hbm, rbuf, ones_v, acc):
        c = lax.axis_index("c")
        s = lax.axis_index("s")
        wid = c * 16 + s
        base = s * RPT
        pltpu.sync_copy(zero_hbm, acc.at[pl.ds(base, RPT)])
        pltpu.sync_copy(ones_hbm, ones_v)
        plsc.subcore_barrier()

        @pl.loop(0, NCHUNK)
        def _(q):
            pltpu.sync_copy(
                ridx_hbm.at[pl.ds((wid * NCHUNK + q) * CB, CB)], rbuf)

            @pl.loop(0, CB)
            def _(j):
                pltpu.sync_copy(ones_v, acc.at[rbuf.at[j]], add=True)

        plsc.subcore_barrier()
        pltpu.sync_copy(acc.at[pl.ds(base, RPT)],
                        out_hbm.at[pl.ds(c * NPAD + base, RPT)])

    return deg


_deg16 = _make_deg()


def _prep_body(x_ref, w_ref, da_ref, db_ref, dis_ref, g0_ref):
    deg = da_ref[:, :1] + db_ref[:, :1] + 1.0
    dis = lax.rsqrt(deg)
    dis_ref[...] = dis
    g0_ref[...] = dis * jnp.dot(x_ref[...], w_ref[...],
                                preferred_element_type=jnp.float32)


def _prep(xp, w1, deg2):
    return pl.pallas_call(
        _prep_body,
        grid=(NRB,),
        in_specs=[
            pl.BlockSpec((RBLK, D), lambda i: (i, 0)),
            pl.BlockSpec((D, D), lambda i: (0, 0)),
            pl.BlockSpec((RBLK, DDEG), lambda i: (i, 0)),
            pl.BlockSpec((RBLK, DDEG), lambda i: (i + NRB, 0)),
        ],
        out_specs=[
            pl.BlockSpec((RBLK, 1), lambda i: (i, 0)),
            pl.BlockSpec((RBLK, D), lambda i: (i, 0)),
        ],
        out_shape=[
            jax.ShapeDtypeStruct((NPAD, 1), jnp.float32),
            jax.ShapeDtypeStruct((NPAD, D), jnp.float32),
        ],
    )(xp, w1, deg2, deg2)


def _layer_body(final, aa_ref, ab_ref, g_ref, dis_ref, w_ref, b_ref, o_ref):
    t = dis_ref[...] * (aa_ref[...] + ab_ref[...] + g_ref[...]) + b_ref[...]
    if final:
        o_ref[...] = t
    else:
        h = jnp.maximum(t, 0.0)
        o_ref[...] = dis_ref[...] * jnp.dot(
            h, w_ref[...], preferred_element_type=jnp.float32)


def _layer(a2, g, dis, w_next, b, final):
    nrows = N if final else NPAD
    return pl.pallas_call(
        functools.partial(_layer_body, final),
        grid=(NRB,),
        in_specs=[
            pl.BlockSpec((RBLK, D), lambda i: (i, 0)),
            pl.BlockSpec((RBLK, D), lambda i: (i + NRB, 0)),
            pl.BlockSpec((RBLK, D), lambda i: (i, 0)),
            pl.BlockSpec((RBLK, 1), lambda i: (i, 0)),
            pl.BlockSpec((D, D), lambda i: (0, 0)),
            pl.BlockSpec((1, D), lambda i: (0, 0)),
        ],
        out_specs=pl.BlockSpec((RBLK, D), lambda i: (i, 0)),
        out_shape=jax.ShapeDtypeStruct((nrows, D), jnp.float32),
    )(a2, a2, g, dis, w_next, b)


def kernel(x, edge_index, W1, b1, W2, b2, W3, b3):
    xp = jnp.pad(x, ((0, NPAD - N), (0, 0)))
    row = edge_index[0].astype(jnp.int32)
    col = edge_index[1].astype(jnp.int32)
    pad = SLOTS - E
    pr = N + jnp.arange(pad, dtype=jnp.int32) % (NPAD - N)
    pc = jnp.arange(pad, dtype=jnp.int32) % N
    ridx = jnp.concatenate([row, pr]).reshape(NT * NBLK, B)
    cidx = jnp.concatenate([col, pc]).reshape(NT * NBLK, B)
    zeros_d = jnp.zeros((RPT, D), jnp.float32)
    ones_b = jnp.ones((B, DDEG), jnp.float32)
    deg2 = _deg16(ridx, zeros_d, ones_b)
    dis, g0 = _prep(xp, W1, deg2)

    a1 = _agg_feat(g0, cidx, ridx, zeros_d)
    g1 = _layer(a1, g0, dis, W2, b1.reshape(1, D), final=False)
    a2 = _agg_feat(g1, cidx, ridx, zeros_d)
    g2 = _layer(a2, g1, dis, W3, b2.reshape(1, D), final=False)
    a3 = _agg_feat(g2, cidx, ridx, zeros_d)
    return _layer(a3, g2, dis, W3, b3.reshape(1, D), final=True)


# speedup vs baseline: 21.2201x; 1.0135x over previous
import functools

import jax
import jax.numpy as jnp
from jax import lax
from jax.experimental import pallas as pl
from jax.experimental.pallas import tpu as pltpu
from jax.experimental.pallas import tpu_sc as plsc

N = 10000
D = 128
E = 320000
NPAD = 10240
NT = 32
B = 128
NBLK = 80
SLOTS = NT * NBLK * B
RPT = NPAD // 16
CB = 40
NCHUNK = NBLK // CB
RBLK = 2048
NRB = NPAD // RBLK


def _make_agg(d):
    mesh = plsc.VectorSubcoreMesh(core_axis_name="c", subcore_axis_name="s")

    @functools.partial(
        pl.kernel,
        out_type=jax.ShapeDtypeStruct((2 * NPAD, d), jnp.float32),
        mesh=mesh,
        scratch_types=[
            pltpu.VMEM((CB, B), jnp.int32),
            pltpu.VMEM((CB, B), jnp.int32),
            pltpu.VMEM((B, d), jnp.float32),
            pltpu.VMEM((B, d), jnp.float32),
            pltpu.MemorySpace.VMEM_SHARED((NPAD, d), jnp.float32),
            pltpu.SemaphoreType.DMA,
            pltpu.SemaphoreType.DMA,
            pltpu.SemaphoreType.DMA,
            pltpu.SemaphoreType.DMA,
        ],
    )
    def agg(g_hbm, cidx_hbm, ridx_hbm, zero_hbm, out_hbm,
            cbuf, rbuf, rows0, rows1, acc_sh, gs0, gs1, ss0, ss1):
        c = lax.axis_index("c")
        s = lax.axis_index("s")
        wid = c * 16 + s
        base = s * RPT
        pltpu.sync_copy(cidx_hbm.at[pl.ds(wid * NBLK, CB)], cbuf)
        pltpu.sync_copy(ridx_hbm.at[pl.ds(wid * NBLK, CB)], rbuf)
        pltpu.async_copy(g_hbm.at[cbuf.at[0]], rows0, gs0)
        pltpu.async_copy(g_hbm.at[cbuf.at[1]], rows1, gs1)
        pltpu.sync_copy(zero_hbm, acc_sh.at[pl.ds(base, RPT)])
        plsc.subcore_barrier()

        for q in range(NCHUNK):
            if q > 0:
                off = wid * NBLK + q * CB
                pltpu.sync_copy(cidx_hbm.at[pl.ds(off, CB)], cbuf)
                pltpu.sync_copy(ridx_hbm.at[pl.ds(off, CB)], rbuf)
                pltpu.async_copy(g_hbm.at[cbuf.at[0]], rows0, gs0)
                pltpu.async_copy(g_hbm.at[cbuf.at[1]], rows1, gs1)

            @pl.loop(0, CB, step=2)
            def _(j):
                pltpu.make_async_copy(g_hbm.at[cbuf.at[j]], rows0, gs0).wait()
                s0 = pltpu.async_copy(rows0, acc_sh.at[rbuf.at[j]], ss0,
                                      add=True)
                pltpu.make_async_copy(g_hbm.at[cbuf.at[j + 1]], rows1,
                                      gs1).wait()
                s1 = pltpu.async_copy(rows1, acc_sh.at[rbuf.at[j + 1]], ss1,
                                      add=True)
                s0.wait()

                @pl.when(j + 2 < CB)
                def _():
                    pltpu.async_copy(g_hbm.at[cbuf.at[j + 2]], rows0, gs0)

                s1.wait()

                @pl.when(j + 2 < CB)
                def _():
                    pltpu.async_copy(g_hbm.at[cbuf.at[j + 3]], rows1, gs1)

        plsc.subcore_barrier()
        pltpu.sync_copy(acc_sh.at[pl.ds(base, RPT)],
                        out_hbm.at[pl.ds(c * NPAD + base, RPT)])

    return agg


_agg_feat = _make_agg(D)

DDEG = D


def _make_deg():
    mesh = plsc.VectorSubcoreMesh(core_axis_name="c", subcore_axis_name="s")

    @functools.partial(
        pl.kernel,
        out_type=jax.ShapeDtypeStruct((2 * NPAD, DDEG), jnp.float32),
        mesh=mesh,
        scratch_types=[
            pltpu.VMEM((CB, B), jnp.int32),
            pltpu.VMEM((B, DDEG), jnp.float32),
            pltpu.MemorySpace.VMEM_SHARED((NPAD, DDEG), jnp.float32),
        ],
    )
    def deg(ridx_hbm, zero_hbm, ones_hbm, out_hbm, rbuf, ones_v, acc):
        c = lax.axis_index("c")
        s = lax.axis_index("s")
        wid = c * 16 + s
        base = s * RPT
        pltpu.sync_copy(zero_hbm, acc.at[pl.ds(base, RPT)])
        pltpu.sync_copy(ones_hbm, ones_v)
        plsc.subcore_barrier()

        @pl.loop(0, NCHUNK)
        def _(q):
            pltpu.sync_copy(
                ridx_hbm.at[pl.ds((wid * NCHUNK + q) * CB, CB)], rbuf)

            @pl.loop(0, CB)
            def _(j):
                pltpu.sync_copy(ones_v, acc.at[rbuf.at[j]], add=True)

        plsc.subcore_barrier()
        pltpu.sync_copy(acc.at[pl.ds(base, RPT)],
                        out_hbm.at[pl.ds(c * NPAD + base, RPT)])

    return deg


_deg16 = _make_deg()


def _prep_body(x_ref, w_ref, da_ref, db_ref, dis_ref, g0_ref):
    deg = da_ref[:, :1] + db_ref[:, :1] + 1.0
    dis = lax.rsqrt(deg)
    dis_ref[...] = dis
    g0_ref[...] = dis * jnp.dot(x_ref[...], w_ref[...],
                                preferred_element_type=jnp.float32)


def _prep(xp, w1, deg2):
    return pl.pallas_call(
        _prep_body,
        grid=(NRB,),
        in_specs=[
            pl.BlockSpec((RBLK, D), lambda i: (i, 0)),
            pl.BlockSpec((D, D), lambda i: (0, 0)),
            pl.BlockSpec((RBLK, DDEG), lambda i: (i, 0)),
            pl.BlockSpec((RBLK, DDEG), lambda i: (i + NRB, 0)),
        ],
        out_specs=[
            pl.BlockSpec((RBLK, 1), lambda i: (i, 0)),
            pl.BlockSpec((RBLK, D), lambda i: (i, 0)),
        ],
        out_shape=[
            jax.ShapeDtypeStruct((NPAD, 1), jnp.float32),
            jax.ShapeDtypeStruct((NPAD, D), jnp.float32),
        ],
    )(xp, w1, deg2, deg2)


def _layer_body(final, aa_ref, ab_ref, g_ref, dis_ref, w_ref, b_ref, o_ref):
    t = dis_ref[...] * (aa_ref[...] + ab_ref[...] + g_ref[...]) + b_ref[...]
    if final:
        o_ref[...] = t
    else:
        h = jnp.maximum(t, 0.0)
        o_ref[...] = dis_ref[...] * jnp.dot(
            h, w_ref[...], preferred_element_type=jnp.float32)


def _layer(a2, g, dis, w_next, b, final):
    nrows = N if final else NPAD
    return pl.pallas_call(
        functools.partial(_layer_body, final),
        grid=(NRB,),
        in_specs=[
            pl.BlockSpec((RBLK, D), lambda i: (i, 0)),
            pl.BlockSpec((RBLK, D), lambda i: (i + NRB, 0)),
            pl.BlockSpec((RBLK, D), lambda i: (i, 0)),
            pl.BlockSpec((RBLK, 1), lambda i: (i, 0)),
            pl.BlockSpec((D, D), lambda i: (0, 0)),
            pl.BlockSpec((1, D), lambda i: (0, 0)),
        ],
        out_specs=pl.BlockSpec((RBLK, D), lambda i: (i, 0)),
        out_shape=jax.ShapeDtypeStruct((nrows, D), jnp.float32),
    )(a2, a2, g, dis, w_next, b)


def kernel(x, edge_index, W1, b1, W2, b2, W3, b3):
    xp = jnp.pad(x, ((0, NPAD - N), (0, 0)))
    row = edge_index[0].astype(jnp.int32)
    col = edge_index[1].astype(jnp.int32)
    pad = SLOTS - E
    pr = N + jnp.arange(pad, dtype=jnp.int32) % (NPAD - N)
    pc = jnp.arange(pad, dtype=jnp.int32) % N
    ridx = jnp.concatenate([row, pr]).reshape(NT * NBLK, B)
    cidx = jnp.concatenate([col, pc]).reshape(NT * NBLK, B)
    zeros_d = jnp.zeros((RPT, D), jnp.float32)
    ones_b = jnp.ones((B, DDEG), jnp.float32)
    deg2 = _deg16(ridx, zeros_d, ones_b)
    dis, g0 = _prep(xp, W1, deg2)

    a1 = _agg_feat(g0, cidx, ridx, zeros_d)
    g1 = _layer(a1, g0, dis, W2, b1.reshape(1, D), final=False)
    a2 = _agg_feat(g1, cidx, ridx, zeros_d)
    g2 = _layer(a2, g1, dis, W3, b2.reshape(1, D), final=False)
    a3 = _agg_feat(g2, cidx, ridx, zeros_d)
    return _layer(a3, g2, dis, W3, b3.reshape(1, D), final=True)


# speedup vs baseline: 21.3432x; 1.0058x over previous
import functools

import jax
import jax.numpy as jnp
from jax import lax
from jax.experimental import pallas as pl
from jax.experimental.pallas import tpu as pltpu
from jax.experimental.pallas import tpu_sc as plsc

N = 10000
D = 128
E = 320000
NPAD = 10240
NT = 32
B = 128
NBLK = 80
SLOTS = NT * NBLK * B
RPT = NPAD // 16
CB = 40
NCHUNK = NBLK // CB
RBLK = 5120
NRB = NPAD // RBLK


def _make_agg(d):
    mesh = plsc.VectorSubcoreMesh(core_axis_name="c", subcore_axis_name="s")

    @functools.partial(
        pl.kernel,
        out_type=jax.ShapeDtypeStruct((2 * NPAD, d), jnp.float32),
        mesh=mesh,
        scratch_types=[
            pltpu.VMEM((CB, B), jnp.int32),
            pltpu.VMEM((CB, B), jnp.int32),
            pltpu.VMEM((B, d), jnp.float32),
            pltpu.VMEM((B, d), jnp.float32),
            pltpu.MemorySpace.VMEM_SHARED((NPAD, d), jnp.float32),
            pltpu.SemaphoreType.DMA,
            pltpu.SemaphoreType.DMA,
            pltpu.SemaphoreType.DMA,
            pltpu.SemaphoreType.DMA,
        ],
    )
    def agg(g_hbm, cidx_hbm, ridx_hbm, zero_hbm, out_hbm,
            cbuf, rbuf, rows0, rows1, acc_sh, gs0, gs1, ss0, ss1):
        c = lax.axis_index("c")
        s = lax.axis_index("s")
        wid = c * 16 + s
        base = s * RPT
        pltpu.sync_copy(cidx_hbm.at[pl.ds(wid * NBLK, CB)], cbuf)
        pltpu.sync_copy(ridx_hbm.at[pl.ds(wid * NBLK, CB)], rbuf)
        pltpu.async_copy(g_hbm.at[cbuf.at[0]], rows0, gs0)
        pltpu.async_copy(g_hbm.at[cbuf.at[1]], rows1, gs1)
        pltpu.sync_copy(zero_hbm, acc_sh.at[pl.ds(base, RPT)])
        plsc.subcore_barrier()

        for q in range(NCHUNK):
            if q > 0:
                off = wid * NBLK + q * CB
                pltpu.sync_copy(cidx_hbm.at[pl.ds(off, CB)], cbuf)
                pltpu.sync_copy(ridx_hbm.at[pl.ds(off, CB)], rbuf)
                pltpu.async_copy(g_hbm.at[cbuf.at[0]], rows0, gs0)
                pltpu.async_copy(g_hbm.at[cbuf.at[1]], rows1, gs1)

            @pl.loop(0, CB, step=2)
            def _(j):
                pltpu.make_async_copy(g_hbm.at[cbuf.at[j]], rows0, gs0).wait()
                s0 = pltpu.async_copy(rows0, acc_sh.at[rbuf.at[j]], ss0,
                                      add=True)
                pltpu.make_async_copy(g_hbm.at[cbuf.at[j + 1]], rows1,
                                      gs1).wait()
                s1 = pltpu.async_copy(rows1, acc_sh.at[rbuf.at[j + 1]], ss1,
                                      add=True)
                s0.wait()

                @pl.when(j + 2 < CB)
                def _():
                    pltpu.async_copy(g_hbm.at[cbuf.at[j + 2]], rows0, gs0)

                s1.wait()

                @pl.when(j + 2 < CB)
                def _():
                    pltpu.async_copy(g_hbm.at[cbuf.at[j + 3]], rows1, gs1)

        plsc.subcore_barrier()
        pltpu.sync_copy(acc_sh.at[pl.ds(base, RPT)],
                        out_hbm.at[pl.ds(c * NPAD + base, RPT)])

    return agg


_agg_feat = _make_agg(D)

DDEG = D


def _make_deg():
    mesh = plsc.VectorSubcoreMesh(core_axis_name="c", subcore_axis_name="s")

    @functools.partial(
        pl.kernel,
        out_type=jax.ShapeDtypeStruct((2 * NPAD, DDEG), jnp.float32),
        mesh=mesh,
        scratch_types=[
            pltpu.VMEM((CB, B), jnp.int32),
            pltpu.VMEM((B, DDEG), jnp.float32),
            pltpu.MemorySpace.VMEM_SHARED((NPAD, DDEG), jnp.float32),
        ],
    )
    def deg(ridx_hbm, zero_hbm, ones_hbm, out_hbm, rbuf, ones_v, acc):
        c = lax.axis_index("c")
        s = lax.axis_index("s")
        wid = c * 16 + s
        base = s * RPT
        pltpu.sync_copy(zero_hbm, acc.at[pl.ds(base, RPT)])
        pltpu.sync_copy(ones_hbm, ones_v)
        plsc.subcore_barrier()

        @pl.loop(0, NCHUNK)
        def _(q):
            pltpu.sync_copy(
                ridx_hbm.at[pl.ds((wid * NCHUNK + q) * CB, CB)], rbuf)

            @pl.loop(0, CB)
            def _(j):
                pltpu.sync_copy(ones_v, acc.at[rbuf.at[j]], add=True)

        plsc.subcore_barrier()
        pltpu.sync_copy(acc.at[pl.ds(base, RPT)],
                        out_hbm.at[pl.ds(c * NPAD + base, RPT)])

    return deg


_deg16 = _make_deg()


def _prep_body(x_ref, w_ref, da_ref, db_ref, dis_ref, g0_ref):
    deg = da_ref[:, :1] + db_ref[:, :1] + 1.0
    dis = lax.rsqrt(deg)
    dis_ref[...] = dis
    g0_ref[...] = dis * jnp.dot(x_ref[...], w_ref[...],
                                preferred_element_type=jnp.float32)


def _prep(xp, w1, deg2):
    return pl.pallas_call(
        _prep_body,
        grid=(NRB,),
        in_specs=[
            pl.BlockSpec((RBLK, D), lambda i: (i, 0)),
            pl.BlockSpec((D, D), lambda i: (0, 0)),
            pl.BlockSpec((RBLK, DDEG), lambda i: (i, 0)),
            pl.BlockSpec((RBLK, DDEG), lambda i: (i + NRB, 0)),
        ],
        out_specs=[
            pl.BlockSpec((RBLK, 1), lambda i: (i, 0)),
            pl.BlockSpec((RBLK, D), lambda i: (i, 0)),
        ],
        out_shape=[
            jax.ShapeDtypeStruct((NPAD, 1), jnp.float32),
            jax.ShapeDtypeStruct((NPAD, D), jnp.float32),
        ],
    )(xp, w1, deg2, deg2)


def _layer_body(final, aa_ref, ab_ref, g_ref, dis_ref, w_ref, b_ref, o_ref):
    t = dis_ref[...] * (aa_ref[...] + ab_ref[...] + g_ref[...]) + b_ref[...]
    if final:
        o_ref[...] = t
    else:
        h = jnp.maximum(t, 0.0)
        o_ref[...] = dis_ref[...] * jnp.dot(
            h, w_ref[...], preferred_element_type=jnp.float32)


def _layer(a2, g, dis, w_next, b, final):
    nrows = N if final else NPAD
    return pl.pallas_call(
        functools.partial(_layer_body, final),
        grid=(NRB,),
        in_specs=[
            pl.BlockSpec((RBLK, D), lambda i: (i, 0)),
            pl.BlockSpec((RBLK, D), lambda i: (i + NRB, 0)),
            pl.BlockSpec((RBLK, D), lambda i: (i, 0)),
            pl.BlockSpec((RBLK, 1), lambda i: (i, 0)),
            pl.BlockSpec((D, D), lambda i: (0, 0)),
            pl.BlockSpec((1, D), lambda i: (0, 0)),
        ],
        out_specs=pl.BlockSpec((RBLK, D), lambda i: (i, 0)),
        out_shape=jax.ShapeDtypeStruct((nrows, D), jnp.float32),
    )(a2, a2, g, dis, w_next, b)


def kernel(x, edge_index, W1, b1, W2, b2, W3, b3):
    xp = jnp.pad(x, ((0, NPAD - N), (0, 0)))
    row = edge_index[0].astype(jnp.int32)
    col = edge_index[1].astype(jnp.int32)
    pad = SLOTS - E
    pr = N + jnp.arange(pad, dtype=jnp.int32) % (NPAD - N)
    pc = jnp.arange(pad, dtype=jnp.int32) % N
    ridx = jnp.concatenate([row, pr]).reshape(NT * NBLK, B)
    cidx = jnp.concatenate([col, pc]).reshape(NT * NBLK, B)
    zeros_d = jnp.zeros((RPT, D), jnp.float32)
    ones_b = jnp.ones((B, DDEG), jnp.float32)
    deg2 = _deg16(ridx, zeros_d, ones_b)
    dis, g0 = _prep(xp, W1, deg2)

    a1 = _agg_feat(g0, cidx, ridx, zeros_d)
    g1 = _layer(a1, g0, dis, W2, b1.reshape(1, D), final=False)
    a2 = _agg_feat(g1, cidx, ridx, zeros_d)
    g2 = _layer(a2, g1, dis, W3, b2.reshape(1, D), final=False)
    a3 = _agg_feat(g2, cidx, ridx, zeros_d)
    return _layer(a3, g2, dis, W3, b3.reshape(1, D), final=True)
